# Initial kernel scaffold; baseline (speedup 1.0000x reference)
import jax
import jax.numpy as jnp
from jax.experimental import pallas as pl


def kernel(x, edge_attr, edge_index, params):
    raise NotImplementedError("write your pallas kernel here")


# speedup vs baseline: 2.1266x; 2.1266x over previous
import functools

import jax
import jax.numpy as jnp
from jax import lax
from jax.experimental import pallas as pl
from jax.experimental.pallas import tpu as pltpu
from jax.experimental.pallas import tpu_sc as plsc

H = 128
_RB = 512
_NB = 1000
_CH = 128
_NW = 32


def _proj_body(x_ref, a_ref, b_ref, u_ref, v_ref):
    x = x_ref[...]
    u_ref[...] = jnp.dot(x, a_ref[...], preferred_element_type=jnp.float32)
    v_ref[...] = jnp.dot(x, b_ref[...], preferred_element_type=jnp.float32)


def _proj(x, a, b):
    n = x.shape[0]
    return pl.pallas_call(
        _proj_body,
        grid=(n // _NB,),
        in_specs=[
            pl.BlockSpec((_NB, H), lambda i: (i, 0)),
            pl.BlockSpec((H, H), lambda i: (0, 0)),
            pl.BlockSpec((H, H), lambda i: (0, 0)),
        ],
        out_specs=[
            pl.BlockSpec((_NB, H), lambda i: (i, 0)),
            pl.BlockSpec((_NB, H), lambda i: (i, 0)),
        ],
        out_shape=[
            jax.ShapeDtypeStruct((n, H), jnp.float32),
            jax.ShapeDtypeStruct((n, H), jnp.float32),
        ],
    )(x, a, b)


def _edge_body(g0_ref, g1_ref, e_ref, c_ref, w2_ref, b1_ref, b2_ref, o_ref):
    e = e_ref[...]
    pre = (g0_ref[0] + g1_ref[0] + b1_ref[...]
           + jnp.dot(e, c_ref[...], preferred_element_type=jnp.float32))
    eh = jnp.maximum(pre, 0.0)
    o_ref[...] = (jnp.dot(eh, w2_ref[...], preferred_element_type=jnp.float32)
                  + b2_ref[...] + e)


def _edge_mlp(g, e, c, w2, b1, b2):
    ne = e.shape[0]
    return pl.pallas_call(
        _edge_body,
        grid=(ne // _RB,),
        in_specs=[
            pl.BlockSpec((1, _RB, H), lambda i: (0, i, 0)),
            pl.BlockSpec((1, _RB, H), lambda i: (1, i, 0)),
            pl.BlockSpec((_RB, H), lambda i: (i, 0)),
            pl.BlockSpec((H, H), lambda i: (0, 0)),
            pl.BlockSpec((H, H), lambda i: (0, 0)),
            pl.BlockSpec((1, H), lambda i: (0, 0)),
            pl.BlockSpec((1, H), lambda i: (0, 0)),
        ],
        out_specs=pl.BlockSpec((_RB, H), lambda i: (i, 0)),
        out_shape=jax.ShapeDtypeStruct((ne, H), jnp.float32),
    )(g, g, e, c, w2, b1, b2)


def _node_body(x_ref, a0_ref, a1_ref, wa_ref, wb_ref, w2_ref, b1_ref, b2_ref,
               o_ref):
    x = x_ref[...]
    a = a0_ref[0] + a1_ref[0]
    pre = (jnp.dot(x, wa_ref[...], preferred_element_type=jnp.float32)
           + jnp.dot(a, wb_ref[...], preferred_element_type=jnp.float32)
           + b1_ref[...])
    nh = jnp.maximum(pre, 0.0)
    o_ref[...] = (jnp.dot(nh, w2_ref[...], preferred_element_type=jnp.float32)
                  + b2_ref[...] + x)


def _node_mlp(x, agg, wa, wb, w2, b1, b2):
    n = x.shape[0]
    return pl.pallas_call(
        _node_body,
        grid=(n // _NB,),
        in_specs=[
            pl.BlockSpec((_NB, H), lambda i: (i, 0)),
            pl.BlockSpec((1, _NB, H), lambda i: (0, i, 0)),
            pl.BlockSpec((1, _NB, H), lambda i: (1, i, 0)),
            pl.BlockSpec((H, H), lambda i: (0, 0)),
            pl.BlockSpec((H, H), lambda i: (0, 0)),
            pl.BlockSpec((H, H), lambda i: (0, 0)),
            pl.BlockSpec((1, H), lambda i: (0, 0)),
            pl.BlockSpec((1, H), lambda i: (0, 0)),
        ],
        out_specs=pl.BlockSpec((_NB, H), lambda i: (i, 0)),
        out_shape=jax.ShapeDtypeStruct((n, H), jnp.float32),
    )(x, agg, agg, wa, wb, w2, b1, b2)


def _sc_gather(u, v, src, dst):
    n_edges = src.shape[0]
    per_w = n_edges // _NW
    full = per_w // _CH
    rem = per_w - full * _CH
    mesh = plsc.VectorSubcoreMesh(core_axis_name="c", subcore_axis_name="s")

    @functools.partial(
        pl.kernel,
        mesh=mesh,
        out_type=jax.ShapeDtypeStruct((2, n_edges, H), jnp.float32),
        scratch_types=[
            pltpu.VMEM((_CH,), jnp.int32),
            pltpu.VMEM((_CH,), jnp.int32),
            pltpu.VMEM((_CH, H), jnp.float32),
            pltpu.VMEM((_CH, H), jnp.float32),
            pltpu.SemaphoreType.DMA,
        ],
    )
    def k(u_hbm, v_hbm, src_hbm, dst_hbm, out_hbm, idx_s, idx_d, buf_u, buf_v,
          sem):
        wid = lax.axis_index("s") * 2 + lax.axis_index("c")
        w0 = wid * per_w

        def chunk(base, size):
            pltpu.sync_copy(src_hbm.at[pl.ds(base, size)],
                            idx_s.at[pl.ds(0, size)])
            pltpu.sync_copy(dst_hbm.at[pl.ds(base, size)],
                            idx_d.at[pl.ds(0, size)])
            pltpu.async_copy(u_hbm.at[idx_s.at[pl.ds(0, size)]],
                             buf_u.at[pl.ds(0, size)], sem).wait()
            pltpu.async_copy(v_hbm.at[idx_d.at[pl.ds(0, size)]],
                             buf_v.at[pl.ds(0, size)], sem).wait()
            pltpu.sync_copy(buf_u.at[pl.ds(0, size)],
                            out_hbm.at[0, pl.ds(base, size)])
            pltpu.sync_copy(buf_v.at[pl.ds(0, size)],
                            out_hbm.at[1, pl.ds(base, size)])

        def body(j, carry):
            chunk(w0 + j * _CH, _CH)
            return carry

        lax.fori_loop(0, full, body, 0)
        if rem:
            chunk(w0 + full * _CH, rem)

    return k(u, v, src, dst)


def _sc_scatter(e_new, dst, zeros_nh):
    n_edges = e_new.shape[0]
    n = zeros_nh.shape[0]
    per_w = n_edges // _NW
    full = per_w // _CH
    rem = per_w - full * _CH
    rows_per_tile = n // 16
    mesh = plsc.VectorSubcoreMesh(core_axis_name="c", subcore_axis_name="s")

    @functools.partial(
        pl.kernel,
        mesh=mesh,
        out_type=jax.ShapeDtypeStruct((2, n, H), jnp.float32),
        scratch_types=[
            pltpu.VMEM((_CH,), jnp.int32),
            pltpu.VMEM((_CH, H), jnp.float32),
            pltpu.VMEM_SHARED((n, H), jnp.float32),
        ],
    )
    def k(e_hbm, dst_hbm, z_hbm, out_hbm, idx_d, rows, acc):
        cid = lax.axis_index("c")
        sid = lax.axis_index("s")
        wid = sid * 2 + cid
        w0 = wid * per_w
        r0 = sid * rows_per_tile

        pltpu.sync_copy(z_hbm.at[pl.ds(r0, rows_per_tile)],
                        acc.at[pl.ds(r0, rows_per_tile)])
        plsc.subcore_barrier()

        def chunk(base, size):
            pltpu.sync_copy(dst_hbm.at[pl.ds(base, size)],
                            idx_d.at[pl.ds(0, size)])
            pltpu.sync_copy(e_hbm.at[pl.ds(base, size)],
                            rows.at[pl.ds(0, size)])
            pltpu.sync_copy(rows.at[pl.ds(0, size)],
                            acc.at[idx_d.at[pl.ds(0, size)]], add=True)

        def body(j, carry):
            chunk(w0 + j * _CH, _CH)
            return carry

        lax.fori_loop(0, full, body, 0)
        if rem:
            chunk(w0 + full * _CH, rem)

        plsc.subcore_barrier()
        pltpu.sync_copy(acc.at[pl.ds(r0, rows_per_tile)],
                        out_hbm.at[cid, pl.ds(r0, rows_per_tile)])

    return k(e_new, dst, zeros_nh)


def kernel(x, edge_attr, edge_index, params):
    src = edge_index[0]
    dst = edge_index[1]
    n = x.shape[0]
    n_acc = ((n + 127) // 128) * 128
    zeros_nh = jnp.zeros((n_acc, H), dtype=jnp.float32)

    cx, ce = x, edge_attr
    for p in params:
        a = p['We1'][0:H]
        b = p['We1'][H:2 * H]
        c = p['We1'][2 * H:3 * H]
        b1 = p['be1'].reshape(1, H)
        b2 = p['be2'].reshape(1, H)
        wa = p['Wn1'][0:H]
        wb = p['Wn1'][H:2 * H]
        n1 = p['bn1'].reshape(1, H)
        n2 = p['bn2'].reshape(1, H)

        u, v = _proj(cx, a, b)
        g = _sc_gather(u, v, src, dst)
        ce = _edge_mlp(g, ce, c, p['We2'], b1, b2)
        agg = _sc_scatter(ce, dst, zeros_nh)
        cx = _node_mlp(cx, agg, wa, wb, p['Wn2'], n1, n2)

    return (cx, ce)


# speedup vs baseline: 2.8130x; 1.3228x over previous
import functools

import jax
import jax.numpy as jnp
from jax import lax
from jax.experimental import pallas as pl
from jax.experimental.pallas import tpu as pltpu
from jax.experimental.pallas import tpu_sc as plsc

H = 128
_RB = 512
_NB = 1000
_CH = 128
_NW = 32


def _proj_body(x_ref, a_ref, b_ref, u_ref, v_ref):
    x = x_ref[...]
    u_ref[...] = jnp.dot(x, a_ref[...], preferred_element_type=jnp.float32)
    v_ref[...] = jnp.dot(x, b_ref[...], preferred_element_type=jnp.float32)


def _proj(x, a, b):
    n = x.shape[0]
    return pl.pallas_call(
        _proj_body,
        grid=(n // _NB,),
        in_specs=[
            pl.BlockSpec((_NB, H), lambda i: (i, 0)),
            pl.BlockSpec((H, H), lambda i: (0, 0)),
            pl.BlockSpec((H, H), lambda i: (0, 0)),
        ],
        out_specs=[
            pl.BlockSpec((_NB, H), lambda i: (i, 0)),
            pl.BlockSpec((_NB, H), lambda i: (i, 0)),
        ],
        out_shape=[
            jax.ShapeDtypeStruct((n, H), jnp.float32),
            jax.ShapeDtypeStruct((n, H), jnp.float32),
        ],
    )(x, a, b)


def _edge_body(g0_ref, g1_ref, e_ref, c_ref, w2_ref, b1_ref, b2_ref, o_ref):
    e = e_ref[...]
    pre = (g0_ref[0] + g1_ref[0] + b1_ref[...]
           + jnp.dot(e, c_ref[...], preferred_element_type=jnp.float32))
    eh = jnp.maximum(pre, 0.0)
    o_ref[...] = (jnp.dot(eh, w2_ref[...], preferred_element_type=jnp.float32)
                  + b2_ref[...] + e)


def _edge_mlp(g, e, c, w2, b1, b2):
    ne = e.shape[0]
    return pl.pallas_call(
        _edge_body,
        grid=(ne // _RB,),
        in_specs=[
            pl.BlockSpec((1, _RB, H), lambda i: (0, i, 0)),
            pl.BlockSpec((1, _RB, H), lambda i: (1, i, 0)),
            pl.BlockSpec((_RB, H), lambda i: (i, 0)),
            pl.BlockSpec((H, H), lambda i: (0, 0)),
            pl.BlockSpec((H, H), lambda i: (0, 0)),
            pl.BlockSpec((1, H), lambda i: (0, 0)),
            pl.BlockSpec((1, H), lambda i: (0, 0)),
        ],
        out_specs=pl.BlockSpec((_RB, H), lambda i: (i, 0)),
        out_shape=jax.ShapeDtypeStruct((ne, H), jnp.float32),
    )(g, g, e, c, w2, b1, b2)


def _node_body(x_ref, a0_ref, a1_ref, wa_ref, wb_ref, w2_ref, b1_ref, b2_ref,
               o_ref):
    x = x_ref[...]
    a = a0_ref[0] + a1_ref[0]
    pre = (jnp.dot(x, wa_ref[...], preferred_element_type=jnp.float32)
           + jnp.dot(a, wb_ref[...], preferred_element_type=jnp.float32)
           + b1_ref[...])
    nh = jnp.maximum(pre, 0.0)
    o_ref[...] = (jnp.dot(nh, w2_ref[...], preferred_element_type=jnp.float32)
                  + b2_ref[...] + x)


def _node_mlp(x, agg, wa, wb, w2, b1, b2):
    n = x.shape[0]
    return pl.pallas_call(
        _node_body,
        grid=(n // _NB,),
        in_specs=[
            pl.BlockSpec((_NB, H), lambda i: (i, 0)),
            pl.BlockSpec((1, _NB, H), lambda i: (0, i, 0)),
            pl.BlockSpec((1, _NB, H), lambda i: (1, i, 0)),
            pl.BlockSpec((H, H), lambda i: (0, 0)),
            pl.BlockSpec((H, H), lambda i: (0, 0)),
            pl.BlockSpec((H, H), lambda i: (0, 0)),
            pl.BlockSpec((1, H), lambda i: (0, 0)),
            pl.BlockSpec((1, H), lambda i: (0, 0)),
        ],
        out_specs=pl.BlockSpec((_NB, H), lambda i: (i, 0)),
        out_shape=jax.ShapeDtypeStruct((n, H), jnp.float32),
    )(x, agg, agg, wa, wb, w2, b1, b2)


_RING = 3
_SRING = 2


def _sc_gather(u, v, src, dst):
    n_edges = src.shape[0]
    per_w = n_edges // _NW
    full = per_w // _CH
    rem = per_w - full * _CH
    groups = full // _RING
    assert groups * _RING == full
    mesh = plsc.VectorSubcoreMesh(core_axis_name="c", subcore_axis_name="s")

    @functools.partial(
        pl.kernel,
        mesh=mesh,
        out_type=jax.ShapeDtypeStruct((2, n_edges, H), jnp.float32),
        scratch_types=(
            [pltpu.VMEM((per_w,), jnp.int32)] * 2
            + [pltpu.VMEM((_CH, H), jnp.float32)] * (2 * _RING)
            + [pltpu.SemaphoreType.DMA] * (2 * _RING)
        ),
    )
    def k(u_hbm, v_hbm, src_hbm, dst_hbm, out_hbm, is_all, id_all, *rest):
        bufs_u = rest[0:_RING]
        bufs_v = rest[_RING:2 * _RING]
        sg = rest[2 * _RING:3 * _RING]
        sw = rest[3 * _RING:4 * _RING]
        wid = lax.axis_index("s") * 2 + lax.axis_index("c")
        w0 = wid * per_w

        pltpu.sync_copy(src_hbm.at[pl.ds(w0, per_w)], is_all)
        pltpu.sync_copy(dst_hbm.at[pl.ds(w0, per_w)], id_all)

        def issue_g(c, b):
            s = pl.ds(c * _CH, _CH)
            pltpu.async_copy(u_hbm.at[is_all.at[s]], bufs_u[b], sg[b])
            pltpu.async_copy(v_hbm.at[id_all.at[s]], bufs_v[b], sg[b])

        def wait_g(b):
            pltpu.make_async_copy(u_hbm.at[pl.ds(0, _CH)], bufs_u[b],
                                  sg[b]).wait()
            pltpu.make_async_copy(v_hbm.at[pl.ds(0, _CH)], bufs_v[b],
                                  sg[b]).wait()

        def issue_w(c, b):
            base = w0 + c * _CH
            pltpu.async_copy(bufs_u[b], out_hbm.at[0, pl.ds(base, _CH)], sw[b])
            pltpu.async_copy(bufs_v[b], out_hbm.at[1, pl.ds(base, _CH)], sw[b])

        def wait_w(b):
            pltpu.make_async_copy(bufs_u[b], out_hbm.at[0, pl.ds(0, _CH)],
                                  sw[b]).wait()
            pltpu.make_async_copy(bufs_v[b], out_hbm.at[1, pl.ds(0, _CH)],
                                  sw[b]).wait()

        def group(g, carry):
            for b in range(_RING):
                c = g * _RING + b
                pb = (b - 1) % _RING

                @pl.when(g > 0)
                def _():
                    wait_w(b)

                issue_g(c, b)
                if b == 0:
                    @pl.when(g > 0)
                    def _():
                        wait_g(pb)
                        issue_w(c - 1, pb)
                else:
                    wait_g(pb)
                    issue_w(c - 1, pb)
            return carry

        lax.fori_loop(0, groups, group, 0)
        lb = (full - 1) % _RING
        wait_g(lb)
        issue_w(full - 1, lb)
        for b in range(_RING):
            wait_w(b)
        if rem:
            base = full * _CH
            s = pl.ds(base, rem)
            d = pl.ds(0, rem)
            pltpu.async_copy(u_hbm.at[is_all.at[s]], bufs_u[0].at[d],
                             sg[0]).wait()
            pltpu.async_copy(v_hbm.at[id_all.at[s]], bufs_v[0].at[d],
                             sg[0]).wait()
            pltpu.sync_copy(bufs_u[0].at[d], out_hbm.at[0, pl.ds(w0 + base, rem)])
            pltpu.sync_copy(bufs_v[0].at[d], out_hbm.at[1, pl.ds(w0 + base, rem)])

    return k(u, v, src, dst)


def _sc_scatter(e_new, dst, zeros_nh):
    n_edges = e_new.shape[0]
    n = zeros_nh.shape[0]
    per_w = n_edges // _NW
    full = per_w // _CH
    rem = per_w - full * _CH
    rows_per_tile = n // 16
    mesh = plsc.VectorSubcoreMesh(core_axis_name="c", subcore_axis_name="s")

    groups = full // _SRING
    assert groups * _SRING == full

    @functools.partial(
        pl.kernel,
        mesh=mesh,
        out_type=jax.ShapeDtypeStruct((2, n, H), jnp.float32),
        scratch_types=(
            [pltpu.VMEM((_CH,), jnp.int32)] * _SRING
            + [pltpu.VMEM((_CH, H), jnp.float32)] * _SRING
            + [pltpu.VMEM((max(rem, 8),), jnp.int32),
               pltpu.VMEM((max(rem, 8), H), jnp.float32)]
            + [pltpu.VMEM_SHARED((n, H), jnp.float32)]
            + [pltpu.SemaphoreType.DMA] * (2 * _SRING)
        ),
    )
    def k(e_hbm, dst_hbm, z_hbm, out_hbm, *rest):
        idx = rest[0:_SRING]
        rows = rest[_SRING:2 * _SRING]
        idx_r = rest[2 * _SRING]
        rows_r = rest[2 * _SRING + 1]
        acc = rest[2 * _SRING + 2]
        sl = rest[2 * _SRING + 3:3 * _SRING + 3]
        ss = rest[3 * _SRING + 3:4 * _SRING + 3]
        cid = lax.axis_index("c")
        sid = lax.axis_index("s")
        wid = sid * 2 + cid
        w0 = wid * per_w
        r0 = sid * rows_per_tile

        pltpu.sync_copy(z_hbm.at[pl.ds(r0, rows_per_tile)],
                        acc.at[pl.ds(r0, rows_per_tile)])
        plsc.subcore_barrier()

        def issue_l(c, b):
            base = w0 + c * _CH
            pltpu.async_copy(dst_hbm.at[pl.ds(base, _CH)], idx[b], sl[b])
            pltpu.async_copy(e_hbm.at[pl.ds(base, _CH)], rows[b], sl[b])

        def wait_l(b):
            pltpu.make_async_copy(dst_hbm.at[pl.ds(0, _CH)], idx[b],
                                  sl[b]).wait()
            pltpu.make_async_copy(e_hbm.at[pl.ds(0, _CH)], rows[b],
                                  sl[b]).wait()

        def issue_s(b):
            pltpu.async_copy(rows[b], acc.at[idx[b]], ss[b], add=True)

        def wait_s(b):
            pltpu.make_async_copy(rows[b], acc.at[pl.ds(0, _CH)], ss[b]).wait()

        def group(g, carry):
            for b in range(_SRING):
                c = g * _SRING + b
                pb = (b - 1) % _SRING

                @pl.when(g > 0)
                def _():
                    wait_s(b)

                issue_l(c, b)
                if b == 0:
                    @pl.when(g > 0)
                    def _():
                        wait_l(pb)
                        issue_s(pb)
                else:
                    wait_l(pb)
                    issue_s(pb)
            return carry

        lax.fori_loop(0, groups, group, 0)
        lb = (full - 1) % _SRING
        wait_l(lb)
        issue_s(lb)
        for b in range(_SRING):
            wait_s(b)
        if rem:
            base = w0 + full * _CH
            pltpu.sync_copy(dst_hbm.at[pl.ds(base, rem)], idx_r)
            pltpu.sync_copy(e_hbm.at[pl.ds(base, rem)], rows_r)
            pltpu.sync_copy(rows_r, acc.at[idx_r], add=True)

        plsc.subcore_barrier()
        pltpu.sync_copy(acc.at[pl.ds(r0, rows_per_tile)],
                        out_hbm.at[cid, pl.ds(r0, rows_per_tile)])

    return k(e_new, dst, zeros_nh)


def kernel(x, edge_attr, edge_index, params):
    src = edge_index[0]
    dst = edge_index[1]
    n = x.shape[0]
    n_acc = ((n + 127) // 128) * 128
    zeros_nh = jnp.zeros((n_acc, H), dtype=jnp.float32)

    cx, ce = x, edge_attr
    for p in params:
        a = p['We1'][0:H]
        b = p['We1'][H:2 * H]
        c = p['We1'][2 * H:3 * H]
        b1 = p['be1'].reshape(1, H)
        b2 = p['be2'].reshape(1, H)
        wa = p['Wn1'][0:H]
        wb = p['Wn1'][H:2 * H]
        n1 = p['bn1'].reshape(1, H)
        n2 = p['bn2'].reshape(1, H)

        u, v = _proj(cx, a, b)
        g = _sc_gather(u, v, src, dst)
        ce = _edge_mlp(g, ce, c, p['We2'], b1, b2)
        agg = _sc_scatter(ce, dst, zeros_nh)
        cx = _node_mlp(cx, agg, wa, wb, p['Wn2'], n1, n2)

    return (cx, ce)


# speedup vs baseline: 3.1087x; 1.1051x over previous
import functools

import jax
import jax.numpy as jnp
from jax import lax
from jax.experimental import pallas as pl
from jax.experimental.pallas import tpu as pltpu
from jax.experimental.pallas import tpu_sc as plsc

H = 128
_RB = 512
_NB = 1000
_CH = 128
_NW = 32


def _proj_body(x_ref, a_ref, b_ref, u_ref, v_ref):
    x = x_ref[...]
    u_ref[...] = jnp.dot(x, a_ref[...], preferred_element_type=jnp.float32)
    v_ref[...] = jnp.dot(x, b_ref[...], preferred_element_type=jnp.float32)


def _proj(x, a, b):
    n = x.shape[0]
    return pl.pallas_call(
        _proj_body,
        grid=(n // _NB,),
        in_specs=[
            pl.BlockSpec((_NB, H), lambda i: (i, 0)),
            pl.BlockSpec((H, H), lambda i: (0, 0)),
            pl.BlockSpec((H, H), lambda i: (0, 0)),
        ],
        out_specs=[
            pl.BlockSpec((_NB, H), lambda i: (i, 0)),
            pl.BlockSpec((_NB, H), lambda i: (i, 0)),
        ],
        out_shape=[
            jax.ShapeDtypeStruct((n, H), jnp.float32),
            jax.ShapeDtypeStruct((n, H), jnp.float32),
        ],
    )(x, a, b)


def _edge_body(g_ref, e_ref, c_ref, w2_ref, b1_ref, b2_ref, o_ref):
    e = e_ref[...]
    pre = (g_ref[...] + b1_ref[...]
           + jnp.dot(e, c_ref[...], preferred_element_type=jnp.float32))
    eh = jnp.maximum(pre, 0.0)
    o_ref[...] = (jnp.dot(eh, w2_ref[...], preferred_element_type=jnp.float32)
                  + b2_ref[...] + e)


def _edge_mlp(g, e, c, w2, b1, b2):
    ne = e.shape[0]
    return pl.pallas_call(
        _edge_body,
        grid=(ne // _RB,),
        in_specs=[
            pl.BlockSpec((_RB, H), lambda i: (i, 0)),
            pl.BlockSpec((_RB, H), lambda i: (i, 0)),
            pl.BlockSpec((H, H), lambda i: (0, 0)),
            pl.BlockSpec((H, H), lambda i: (0, 0)),
            pl.BlockSpec((1, H), lambda i: (0, 0)),
            pl.BlockSpec((1, H), lambda i: (0, 0)),
        ],
        out_specs=pl.BlockSpec((_RB, H), lambda i: (i, 0)),
        out_shape=jax.ShapeDtypeStruct((ne, H), jnp.float32),
    )(g, e, c, w2, b1, b2)


def _node_body(x_ref, a0_ref, a1_ref, wa_ref, wb_ref, w2_ref, b1_ref, b2_ref,
               o_ref):
    x = x_ref[...]
    a = a0_ref[0] + a1_ref[0]
    pre = (jnp.dot(x, wa_ref[...], preferred_element_type=jnp.float32)
           + jnp.dot(a, wb_ref[...], preferred_element_type=jnp.float32)
           + b1_ref[...])
    nh = jnp.maximum(pre, 0.0)
    o_ref[...] = (jnp.dot(nh, w2_ref[...], preferred_element_type=jnp.float32)
                  + b2_ref[...] + x)


def _node_mlp(x, agg, wa, wb, w2, b1, b2):
    n = x.shape[0]
    return pl.pallas_call(
        _node_body,
        grid=(n // _NB,),
        in_specs=[
            pl.BlockSpec((_NB, H), lambda i: (i, 0)),
            pl.BlockSpec((1, _NB, H), lambda i: (0, i, 0)),
            pl.BlockSpec((1, _NB, H), lambda i: (1, i, 0)),
            pl.BlockSpec((H, H), lambda i: (0, 0)),
            pl.BlockSpec((H, H), lambda i: (0, 0)),
            pl.BlockSpec((H, H), lambda i: (0, 0)),
            pl.BlockSpec((1, H), lambda i: (0, 0)),
            pl.BlockSpec((1, H), lambda i: (0, 0)),
        ],
        out_specs=pl.BlockSpec((_NB, H), lambda i: (i, 0)),
        out_shape=jax.ShapeDtypeStruct((n, H), jnp.float32),
    )(x, agg, agg, wa, wb, w2, b1, b2)


_RING = 3
_SRING = 2


def _sc_gather(u, v, src, dst):
    n_edges = src.shape[0]
    per_w = n_edges // _NW
    full = per_w // _CH
    rem = per_w - full * _CH
    groups = full // _RING
    assert groups * _RING == full
    mesh = plsc.VectorSubcoreMesh(core_axis_name="c", subcore_axis_name="s")

    @functools.partial(
        pl.kernel,
        mesh=mesh,
        out_type=jax.ShapeDtypeStruct((n_edges, H), jnp.float32),
        scratch_types=(
            [pltpu.VMEM((per_w,), jnp.int32)] * 2
            + [pltpu.VMEM((_CH, H), jnp.float32)] * (2 * _RING)
            + [pltpu.SemaphoreType.DMA] * (2 * _RING)
        ),
    )
    def k(u_hbm, v_hbm, src_hbm, dst_hbm, out_hbm, is_all, id_all, *rest):
        bufs_u = rest[0:_RING]
        bufs_v = rest[_RING:2 * _RING]
        sg = rest[2 * _RING:3 * _RING]
        sw = rest[3 * _RING:4 * _RING]
        wid = lax.axis_index("s") * 2 + lax.axis_index("c")
        w0 = wid * per_w

        pltpu.sync_copy(src_hbm.at[pl.ds(w0, per_w)], is_all)
        pltpu.sync_copy(dst_hbm.at[pl.ds(w0, per_w)], id_all)

        def issue_g(c, b):
            s = pl.ds(c * _CH, _CH)
            pltpu.async_copy(u_hbm.at[is_all.at[s]], bufs_u[b], sg[b])
            pltpu.async_copy(v_hbm.at[id_all.at[s]], bufs_v[b], sg[b])

        def wait_g(b):
            pltpu.make_async_copy(u_hbm.at[pl.ds(0, _CH)], bufs_u[b],
                                  sg[b]).wait()
            pltpu.make_async_copy(v_hbm.at[pl.ds(0, _CH)], bufs_v[b],
                                  sg[b]).wait()

        def add_uv(b, nrows):
            bu, bv = bufs_u[b], bufs_v[b]

            def row(r, carry):
                for cc in range(H // 16):
                    cs = pl.ds(cc * 16, 16)
                    bu[r, cs] = bu[r, cs] + bv[r, cs]
                return carry

            lax.fori_loop(0, nrows, row, 0)

        def issue_w(c, b):
            base = w0 + c * _CH
            pltpu.async_copy(bufs_u[b], out_hbm.at[pl.ds(base, _CH)], sw[b])

        def wait_w(b):
            pltpu.make_async_copy(bufs_u[b], out_hbm.at[pl.ds(0, _CH)],
                                  sw[b]).wait()

        def group(g, carry):
            for b in range(_RING):
                c = g * _RING + b
                pb = (b - 1) % _RING

                @pl.when(g > 0)
                def _():
                    wait_w(b)

                issue_g(c, b)
                if b == 0:
                    @pl.when(g > 0)
                    def _():
                        wait_g(pb)
                        add_uv(pb, _CH)
                        issue_w(c - 1, pb)
                else:
                    wait_g(pb)
                    add_uv(pb, _CH)
                    issue_w(c - 1, pb)
            return carry

        lax.fori_loop(0, groups, group, 0)
        lb = (full - 1) % _RING
        wait_g(lb)
        add_uv(lb, _CH)
        issue_w(full - 1, lb)
        for b in range(_RING):
            wait_w(b)
        if rem:
            base = full * _CH
            s = pl.ds(base, rem)
            d = pl.ds(0, rem)
            pltpu.async_copy(u_hbm.at[is_all.at[s]], bufs_u[0].at[d],
                             sg[0]).wait()
            pltpu.async_copy(v_hbm.at[id_all.at[s]], bufs_v[0].at[d],
                             sg[0]).wait()
            add_uv(0, rem)
            pltpu.sync_copy(bufs_u[0].at[d], out_hbm.at[pl.ds(w0 + base, rem)])

    return k(u, v, src, dst)


def _sc_scatter(e_new, dst, zeros_nh):
    n_edges = e_new.shape[0]
    n = zeros_nh.shape[0]
    per_w = n_edges // _NW
    full = per_w // _CH
    rem = per_w - full * _CH
    rows_per_tile = n // 16
    mesh = plsc.VectorSubcoreMesh(core_axis_name="c", subcore_axis_name="s")

    groups = full // _SRING
    assert groups * _SRING == full

    @functools.partial(
        pl.kernel,
        mesh=mesh,
        out_type=jax.ShapeDtypeStruct((2, n, H), jnp.float32),
        scratch_types=(
            [pltpu.VMEM((_CH,), jnp.int32)] * _SRING
            + [pltpu.VMEM((_CH, H), jnp.float32)] * _SRING
            + [pltpu.VMEM((max(rem, 8),), jnp.int32),
               pltpu.VMEM((max(rem, 8), H), jnp.float32)]
            + [pltpu.VMEM_SHARED((n, H), jnp.float32)]
            + [pltpu.SemaphoreType.DMA] * (2 * _SRING)
        ),
    )
    def k(e_hbm, dst_hbm, z_hbm, out_hbm, *rest):
        idx = rest[0:_SRING]
        rows = rest[_SRING:2 * _SRING]
        idx_r = rest[2 * _SRING]
        rows_r = rest[2 * _SRING + 1]
        acc = rest[2 * _SRING + 2]
        sl = rest[2 * _SRING + 3:3 * _SRING + 3]
        ss = rest[3 * _SRING + 3:4 * _SRING + 3]
        cid = lax.axis_index("c")
        sid = lax.axis_index("s")
        wid = sid * 2 + cid
        w0 = wid * per_w
        r0 = sid * rows_per_tile

        pltpu.sync_copy(z_hbm.at[pl.ds(r0, rows_per_tile)],
                        acc.at[pl.ds(r0, rows_per_tile)])
        plsc.subcore_barrier()

        def issue_l(c, b):
            base = w0 + c * _CH
            pltpu.async_copy(dst_hbm.at[pl.ds(base, _CH)], idx[b], sl[b])
            pltpu.async_copy(e_hbm.at[pl.ds(base, _CH)], rows[b], sl[b])

        def wait_l(b):
            pltpu.make_async_copy(dst_hbm.at[pl.ds(0, _CH)], idx[b],
                                  sl[b]).wait()
            pltpu.make_async_copy(e_hbm.at[pl.ds(0, _CH)], rows[b],
                                  sl[b]).wait()

        def issue_s(b):
            pltpu.async_copy(rows[b], acc.at[idx[b]], ss[b], add=True)

        def wait_s(b):
            pltpu.make_async_copy(rows[b], acc.at[pl.ds(0, _CH)], ss[b]).wait()

        def group(g, carry):
            for b in range(_SRING):
                c = g * _SRING + b
                pb = (b - 1) % _SRING

                @pl.when(g > 0)
                def _():
                    wait_s(b)

                issue_l(c, b)
                if b == 0:
                    @pl.when(g > 0)
                    def _():
                        wait_l(pb)
                        issue_s(pb)
                else:
                    wait_l(pb)
                    issue_s(pb)
            return carry

        lax.fori_loop(0, groups, group, 0)
        lb = (full - 1) % _SRING
        wait_l(lb)
        issue_s(lb)
        for b in range(_SRING):
            wait_s(b)
        if rem:
            base = w0 + full * _CH
            pltpu.sync_copy(dst_hbm.at[pl.ds(base, rem)], idx_r)
            pltpu.sync_copy(e_hbm.at[pl.ds(base, rem)], rows_r)
            pltpu.sync_copy(rows_r, acc.at[idx_r], add=True)

        plsc.subcore_barrier()
        pltpu.sync_copy(acc.at[pl.ds(r0, rows_per_tile)],
                        out_hbm.at[cid, pl.ds(r0, rows_per_tile)])

    return k(e_new, dst, zeros_nh)


def kernel(x, edge_attr, edge_index, params):
    src = edge_index[0]
    dst = edge_index[1]
    n = x.shape[0]
    n_acc = ((n + 127) // 128) * 128
    zeros_nh = jnp.zeros((n_acc, H), dtype=jnp.float32)

    cx, ce = x, edge_attr
    for p in params:
        a = p['We1'][0:H]
        b = p['We1'][H:2 * H]
        c = p['We1'][2 * H:3 * H]
        b1 = p['be1'].reshape(1, H)
        b2 = p['be2'].reshape(1, H)
        wa = p['Wn1'][0:H]
        wb = p['Wn1'][H:2 * H]
        n1 = p['bn1'].reshape(1, H)
        n2 = p['bn2'].reshape(1, H)

        u, v = _proj(cx, a, b)
        g = _sc_gather(u, v, src, dst)
        ce = _edge_mlp(g, ce, c, p['We2'], b1, b2)
        agg = _sc_scatter(ce, dst, zeros_nh)
        cx = _node_mlp(cx, agg, wa, wb, p['Wn2'], n1, n2)

    return (cx, ce)


# speedup vs baseline: 3.6176x; 1.1637x over previous
import functools

import jax
import jax.numpy as jnp
from jax import lax
from jax.experimental import pallas as pl
from jax.experimental.pallas import tpu as pltpu
from jax.experimental.pallas import tpu_sc as plsc

H = 128
_RB = 640
_NB = 1000
_CH = 128
_CHS = 104
_NW = 32
_RING = 3


def _proj_body(x_ref, a_ref, b_ref, u_ref, v_ref):
    x = x_ref[...]
    u_ref[...] = jnp.dot(x, a_ref[...], preferred_element_type=jnp.float32)
    v_ref[...] = jnp.dot(x, b_ref[...], preferred_element_type=jnp.float32)


def _proj(x, a, b):
    n = x.shape[0]
    return pl.pallas_call(
        _proj_body,
        grid=(n // _NB,),
        in_specs=[
            pl.BlockSpec((_NB, H), lambda i: (i, 0)),
            pl.BlockSpec((H, H), lambda i: (0, 0)),
            pl.BlockSpec((H, H), lambda i: (0, 0)),
        ],
        out_specs=[
            pl.BlockSpec((_NB, H), lambda i: (i, 0)),
            pl.BlockSpec((_NB, H), lambda i: (i, 0)),
        ],
        out_shape=[
            jax.ShapeDtypeStruct((n, H), jnp.float32),
            jax.ShapeDtypeStruct((n, H), jnp.float32),
        ],
    )(x, a, b)


def _edge_body(g_ref, e_ref, c_ref, w2_ref, b1_ref, b2_ref, o_ref):
    e = e_ref[...]
    pre = (g_ref[...] + b1_ref[...]
           + jnp.dot(e, c_ref[...], preferred_element_type=jnp.float32))
    eh = jnp.maximum(pre, 0.0)
    o_ref[...] = (jnp.dot(eh, w2_ref[...], preferred_element_type=jnp.float32)
                  + b2_ref[...] + e)


def _edge_mlp(g, e, e_off_blocks, c, w2, b1, b2):
    ne = g.shape[0]
    return pl.pallas_call(
        _edge_body,
        grid=(ne // _RB,),
        in_specs=[
            pl.BlockSpec((_RB, H), lambda i: (i, 0)),
            pl.BlockSpec((_RB, H), lambda i: (i + e_off_blocks, 0)),
            pl.BlockSpec((H, H), lambda i: (0, 0)),
            pl.BlockSpec((H, H), lambda i: (0, 0)),
            pl.BlockSpec((1, H), lambda i: (0, 0)),
            pl.BlockSpec((1, H), lambda i: (0, 0)),
        ],
        out_specs=pl.BlockSpec((_RB, H), lambda i: (i, 0)),
        out_shape=jax.ShapeDtypeStruct((ne, H), jnp.float32),
    )(g, e, c, w2, b1, b2)


def _node_mlp(x, aggs, wa, wb, w2, b1, b2):
    n = x.shape[0]
    na = len(aggs)

    def body(*refs):
        x_ref = refs[0]
        agg_refs = refs[1:1 + 2 * na]
        wa_ref, wb_ref, w2_ref, b1_ref, b2_ref, o_ref = refs[1 + 2 * na:]
        x = x_ref[...]
        a = agg_refs[0][0]
        for r in agg_refs[1:]:
            a = a + r[0]
        pre = (jnp.dot(x, wa_ref[...], preferred_element_type=jnp.float32)
               + jnp.dot(a, wb_ref[...], preferred_element_type=jnp.float32)
               + b1_ref[...])
        nh = jnp.maximum(pre, 0.0)
        o_ref[...] = (
            jnp.dot(nh, w2_ref[...], preferred_element_type=jnp.float32)
            + b2_ref[...] + x)

    agg_specs = []
    for _ in aggs:
        agg_specs.append(pl.BlockSpec((1, _NB, H), lambda i: (0, i, 0)))
        agg_specs.append(pl.BlockSpec((1, _NB, H), lambda i: (1, i, 0)))
    agg_args = [a for a in aggs for _ in range(2)]
    return pl.pallas_call(
        body,
        grid=(n // _NB,),
        in_specs=(
            [pl.BlockSpec((_NB, H), lambda i: (i, 0))]
            + agg_specs
            + [pl.BlockSpec((H, H), lambda i: (0, 0))] * 3
            + [pl.BlockSpec((1, H), lambda i: (0, 0))] * 2
        ),
        out_specs=pl.BlockSpec((_NB, H), lambda i: (i, 0)),
        out_shape=jax.ShapeDtypeStruct((n, H), jnp.float32),
    )(x, *agg_args, wa, wb, w2, b1, b2)


def _sc_gather(u, v, src, dst):
    n_edges = src.shape[0]
    per_w = n_edges // _NW
    full = per_w // _CH
    rem = per_w - full * _CH
    groups = full // _RING
    assert groups * _RING == full
    mesh = plsc.VectorSubcoreMesh(core_axis_name="c", subcore_axis_name="s")

    @functools.partial(
        pl.kernel,
        mesh=mesh,
        out_type=jax.ShapeDtypeStruct((n_edges, H), jnp.float32),
        scratch_types=(
            [pltpu.VMEM((per_w,), jnp.int32)] * 2
            + [pltpu.VMEM((_CH, H), jnp.float32)] * (2 * _RING)
            + [pltpu.SemaphoreType.DMA] * (2 * _RING)
        ),
    )
    def k(u_hbm, v_hbm, src_hbm, dst_hbm, out_hbm, is_all, id_all, *rest):
        bufs_u = rest[0:_RING]
        bufs_v = rest[_RING:2 * _RING]
        sg = rest[2 * _RING:3 * _RING]
        sw = rest[3 * _RING:4 * _RING]
        wid = lax.axis_index("s") * 2 + lax.axis_index("c")
        w0 = wid * per_w

        pltpu.sync_copy(src_hbm.at[pl.ds(w0, per_w)], is_all)
        pltpu.sync_copy(dst_hbm.at[pl.ds(w0, per_w)], id_all)

        def issue_g(c, b):
            s = pl.ds(c * _CH, _CH)
            pltpu.async_copy(u_hbm.at[is_all.at[s]], bufs_u[b], sg[b])
            pltpu.async_copy(v_hbm.at[id_all.at[s]], bufs_v[b], sg[b])

        def wait_g(b):
            pltpu.make_async_copy(u_hbm.at[pl.ds(0, _CH)], bufs_u[b],
                                  sg[b]).wait()
            pltpu.make_async_copy(v_hbm.at[pl.ds(0, _CH)], bufs_v[b],
                                  sg[b]).wait()

        def add_uv(b, nrows):
            bu, bv = bufs_u[b], bufs_v[b]

            def row(r, carry):
                for cc in range(H // 16):
                    cs = pl.ds(cc * 16, 16)
                    bu[r, cs] = bu[r, cs] + bv[r, cs]
                return carry

            lax.fori_loop(0, nrows, row, 0)

        def issue_w(c, b):
            base = w0 + c * _CH
            pltpu.async_copy(bufs_u[b], out_hbm.at[pl.ds(base, _CH)], sw[b])

        def wait_w(b):
            pltpu.make_async_copy(bufs_u[b], out_hbm.at[pl.ds(0, _CH)],
                                  sw[b]).wait()

        def group(g, carry):
            for b in range(_RING):
                c = g * _RING + b
                pb = (b - 1) % _RING

                @pl.when(g > 0)
                def _():
                    wait_w(b)

                issue_g(c, b)
                if b == 0:
                    @pl.when(g > 0)
                    def _():
                        wait_g(pb)
                        add_uv(pb, _CH)
                        issue_w(c - 1, pb)
                else:
                    wait_g(pb)
                    add_uv(pb, _CH)
                    issue_w(c - 1, pb)
            return carry

        lax.fori_loop(0, groups, group, 0)
        lb = (full - 1) % _RING
        wait_g(lb)
        add_uv(lb, _CH)
        issue_w(full - 1, lb)
        for b in range(_RING):
            wait_w(b)
        if rem:
            base = full * _CH
            s = pl.ds(base, rem)
            d = pl.ds(0, rem)
            pltpu.async_copy(u_hbm.at[is_all.at[s]], bufs_u[0].at[d],
                             sg[0]).wait()
            pltpu.async_copy(v_hbm.at[id_all.at[s]], bufs_v[0].at[d],
                             sg[0]).wait()
            add_uv(0, rem)
            pltpu.sync_copy(bufs_u[0].at[d], out_hbm.at[pl.ds(w0 + base, rem)])

    return k(u, v, src, dst)


def _sc_scatter(parts, dst, zeros_nh, emit_ce=False):
    n = zeros_nh.shape[0]
    rows_per_tile = n // 16
    total = sum(arr.shape[0] for arr, _ in parts)
    mesh = plsc.VectorSubcoreMesh(core_axis_name="c", subcore_axis_name="s")

    meta = []
    for arr, off in parts:
        per_w = arr.shape[0] // _NW
        full = per_w // _CHS
        rem = per_w - full * _CHS
        assert full > 0 and full % _RING == 0
        meta.append((per_w, full, rem, off))
    rmax = max(max(m[2] for m in meta), 8)

    out_type = [jax.ShapeDtypeStruct((2, n, H), jnp.float32)]
    if emit_ce:
        out_type.append(jax.ShapeDtypeStruct((total, H), jnp.float32))

    @functools.partial(
        pl.kernel,
        mesh=mesh,
        out_type=tuple(out_type),
        scratch_types=(
            [pltpu.VMEM((_CHS,), jnp.int32)] * _RING
            + [pltpu.VMEM((_CHS, H), jnp.float32)] * _RING
            + [pltpu.VMEM((rmax,), jnp.int32),
               pltpu.VMEM((rmax, H), jnp.float32)]
            + [pltpu.VMEM_SHARED((n, H), jnp.float32)]
            + [pltpu.SemaphoreType.DMA] * (3 * _RING)
        ),
    )
    def k(*refs):
        np_ = len(parts)
        part_refs = refs[0:np_]
        dst_hbm = refs[np_]
        z_hbm = refs[np_ + 1]
        agg_hbm = refs[np_ + 2]
        pos = np_ + 3
        ce_hbm = refs[pos] if emit_ce else None
        pos += 1 if emit_ce else 0
        rest = refs[pos:]
        idx = rest[0:_RING]
        rows = rest[_RING:2 * _RING]
        idx_r = rest[2 * _RING]
        rows_r = rest[2 * _RING + 1]
        acc = rest[2 * _RING + 2]
        sl = rest[2 * _RING + 3:3 * _RING + 3]
        ss = rest[3 * _RING + 3:4 * _RING + 3]
        sc = rest[4 * _RING + 3:5 * _RING + 3]
        cid = lax.axis_index("c")
        sid = lax.axis_index("s")
        wid = sid * 2 + cid
        r0 = sid * rows_per_tile

        pltpu.sync_copy(z_hbm.at[pl.ds(r0, rows_per_tile)],
                        acc.at[pl.ds(r0, rows_per_tile)])
        plsc.subcore_barrier()

        for pi in range(np_):
            e_hbm = part_refs[pi]
            per_w, full, rem, off = meta[pi]
            w0l = wid * per_w
            groups = full // _RING

            def issue_l(c, b):
                bl = w0l + c * _CHS
                pltpu.async_copy(dst_hbm.at[pl.ds(off + bl, _CHS)], idx[b],
                                 sl[b])
                pltpu.async_copy(e_hbm.at[pl.ds(bl, _CHS)], rows[b], sl[b])

            def wait_l(b):
                pltpu.make_async_copy(dst_hbm.at[pl.ds(0, _CHS)], idx[b],
                                      sl[b]).wait()
                pltpu.make_async_copy(e_hbm.at[pl.ds(0, _CHS)], rows[b],
                                      sl[b]).wait()

            def issue_s(c, b):
                pltpu.async_copy(rows[b], acc.at[idx[b]], ss[b], add=True)
                if emit_ce:
                    base = off + w0l + c * _CHS
                    pltpu.async_copy(rows[b], ce_hbm.at[pl.ds(base, _CHS)],
                                     sc[b])

            def wait_s(b):
                pltpu.make_async_copy(rows[b], acc.at[pl.ds(0, _CHS)],
                                      ss[b]).wait()
                if emit_ce:
                    pltpu.make_async_copy(rows[b], ce_hbm.at[pl.ds(0, _CHS)],
                                          sc[b]).wait()

            def group(g, carry):
                for b in range(_RING):
                    c = g * _RING + b
                    pb = (b - 1) % _RING

                    @pl.when(g > 0)
                    def _():
                        wait_s(b)

                    issue_l(c, b)
                    if b == 0:
                        @pl.when(g > 0)
                        def _():
                            wait_l(pb)
                            issue_s(c - 1, pb)
                    else:
                        wait_l(pb)
                        issue_s(c - 1, pb)
                return carry

            lax.fori_loop(0, groups, group, 0)
            lb = (full - 1) % _RING
            wait_l(lb)
            issue_s(full - 1, lb)
            for b in range(_RING):
                wait_s(b)
            if rem:
                bl = w0l + full * _CHS
                rs = pl.ds(0, rem)
                pltpu.sync_copy(dst_hbm.at[pl.ds(off + bl, rem)],
                                idx_r.at[rs] if rem != rmax else idx_r)
                pltpu.sync_copy(e_hbm.at[pl.ds(bl, rem)],
                                rows_r.at[rs] if rem != rmax else rows_r)
                pltpu.sync_copy(rows_r.at[rs] if rem != rmax else rows_r,
                                acc.at[idx_r.at[rs] if rem != rmax else idx_r],
                                add=True)
                if emit_ce:
                    pltpu.sync_copy(
                        rows_r.at[rs] if rem != rmax else rows_r,
                        ce_hbm.at[pl.ds(off + bl, rem)])

        plsc.subcore_barrier()
        pltpu.sync_copy(acc.at[pl.ds(r0, rows_per_tile)],
                        agg_hbm.at[cid, pl.ds(r0, rows_per_tile)])

    out = k(*[arr for arr, _ in parts], dst, zeros_nh)
    return out if emit_ce else out[0]


def kernel(x, edge_attr, edge_index, params):
    src = edge_index[0]
    dst = edge_index[1]
    n = x.shape[0]
    ne = edge_attr.shape[0]
    el = ne // 2
    n_acc = ((n + 127) // 128) * 128
    zeros_nh = jnp.zeros((n_acc, H), dtype=jnp.float32)
    src_l, src_r = src[:el], src[el:]
    dst_l, dst_r = dst[:el], dst[el:]

    cx = x
    ce_l, ce_r = None, None
    ce_out = None
    for bi, p in enumerate(params):
        a = p['We1'][0:H]
        b = p['We1'][H:2 * H]
        c = p['We1'][2 * H:3 * H]
        w2 = p['We2']
        b1 = p['be1'].reshape(1, H)
        b2 = p['be2'].reshape(1, H)
        wa = p['Wn1'][0:H]
        wb = p['Wn1'][H:2 * H]
        n1 = p['bn1'].reshape(1, H)
        n2 = p['bn2'].reshape(1, H)
        last = bi == len(params) - 1

        u, v = _proj(cx, a, b)
        g_l = _sc_gather(u, v, src_l, dst_l)
        g_r = _sc_gather(u, v, src_r, dst_r)
        if bi == 0:
            e_l = _edge_mlp(g_l, edge_attr, 0, c, w2, b1, b2)
            e_r = _edge_mlp(g_r, edge_attr, el // _RB, c, w2, b1, b2)
        else:
            e_l = _edge_mlp(g_l, ce_l, 0, c, w2, b1, b2)
            e_r = _edge_mlp(g_r, ce_r, 0, c, w2, b1, b2)
        if last:
            agg, ce_out = _sc_scatter([(e_l, 0), (e_r, el)], dst, zeros_nh,
                                      emit_ce=True)
            aggs = [agg]
        else:
            agg_l = _sc_scatter([(e_l, 0)], dst, zeros_nh)
            agg_r = _sc_scatter([(e_r, el)], dst, zeros_nh)
            aggs = [agg_l, agg_r]
        cx = _node_mlp(cx, aggs, wa, wb, p['Wn2'], n1, n2)
        ce_l, ce_r = e_l, e_r

    return (cx, ce_out)


# speedup vs baseline: 4.4512x; 1.2304x over previous
import functools

import jax
import jax.numpy as jnp
from jax import lax
from jax.experimental import pallas as pl
from jax.experimental.pallas import tpu as pltpu
from jax.experimental.pallas import tpu_sc as plsc

H = 128
_RB = 1280
_NB = 1000
_CH = 128
_CHS = 104
_NW = 32
_RING = 3


def _proj_body(x_ref, a_ref, b_ref, u_ref, v_ref):
    x = x_ref[...]
    u_ref[...] = jnp.dot(x, a_ref[...], preferred_element_type=jnp.float32)
    v_ref[...] = jnp.dot(x, b_ref[...], preferred_element_type=jnp.float32)


def _proj(x, a, b):
    n = x.shape[0]
    return pl.pallas_call(
        _proj_body,
        grid=(n // _NB,),
        in_specs=[
            pl.BlockSpec((_NB, H), lambda i: (i, 0)),
            pl.BlockSpec((H, H), lambda i: (0, 0)),
            pl.BlockSpec((H, H), lambda i: (0, 0)),
        ],
        out_specs=[
            pl.BlockSpec((_NB, H), lambda i: (i, 0)),
            pl.BlockSpec((_NB, H), lambda i: (i, 0)),
        ],
        out_shape=[
            jax.ShapeDtypeStruct((n, H), jnp.float32),
            jax.ShapeDtypeStruct((n, H), jnp.float32),
        ],
    )(x, a, b)


def _edge_body(g_ref, e_ref, c_ref, w2_ref, b1_ref, b2_ref, o_ref):
    e = e_ref[...]
    pre = (g_ref[...] + b1_ref[...]
           + jnp.dot(e, c_ref[...], preferred_element_type=jnp.float32))
    eh = jnp.maximum(pre, 0.0)
    o_ref[...] = (jnp.dot(eh, w2_ref[...], preferred_element_type=jnp.float32)
                  + b2_ref[...] + e)


def _edge_mlp(g, e, e_off_blocks, c, w2, b1, b2):
    ne = g.shape[0]
    return pl.pallas_call(
        _edge_body,
        grid=(ne // _RB,),
        in_specs=[
            pl.BlockSpec((_RB, H), lambda i: (i, 0)),
            pl.BlockSpec((_RB, H), lambda i: (i + e_off_blocks, 0)),
            pl.BlockSpec((H, H), lambda i: (0, 0)),
            pl.BlockSpec((H, H), lambda i: (0, 0)),
            pl.BlockSpec((1, H), lambda i: (0, 0)),
            pl.BlockSpec((1, H), lambda i: (0, 0)),
        ],
        out_specs=pl.BlockSpec((_RB, H), lambda i: (i, 0)),
        out_shape=jax.ShapeDtypeStruct((ne, H), jnp.float32),
    )(g, e, c, w2, b1, b2)


def _node_mlp(x, aggs, wa, wb, w2, b1, b2):
    n = x.shape[0]
    na = len(aggs)

    def body(*refs):
        x_ref = refs[0]
        agg_refs = refs[1:1 + 2 * na]
        wa_ref, wb_ref, w2_ref, b1_ref, b2_ref, o_ref = refs[1 + 2 * na:]
        x = x_ref[...]
        a = agg_refs[0][0]
        for r in agg_refs[1:]:
            a = a + r[0]
        pre = (jnp.dot(x, wa_ref[...], preferred_element_type=jnp.float32)
               + jnp.dot(a, wb_ref[...], preferred_element_type=jnp.float32)
               + b1_ref[...])
        nh = jnp.maximum(pre, 0.0)
        o_ref[...] = (
            jnp.dot(nh, w2_ref[...], preferred_element_type=jnp.float32)
            + b2_ref[...] + x)

    agg_specs = []
    for _ in aggs:
        agg_specs.append(pl.BlockSpec((1, _NB, H), lambda i: (0, i, 0)))
        agg_specs.append(pl.BlockSpec((1, _NB, H), lambda i: (1, i, 0)))
    agg_args = [a for a in aggs for _ in range(2)]
    return pl.pallas_call(
        body,
        grid=(n // _NB,),
        in_specs=(
            [pl.BlockSpec((_NB, H), lambda i: (i, 0))]
            + agg_specs
            + [pl.BlockSpec((H, H), lambda i: (0, 0))] * 3
            + [pl.BlockSpec((1, H), lambda i: (0, 0))] * 2
        ),
        out_specs=pl.BlockSpec((_NB, H), lambda i: (i, 0)),
        out_shape=jax.ShapeDtypeStruct((n, H), jnp.float32),
    )(x, *agg_args, wa, wb, w2, b1, b2)


def _sc_gather(u, v, src, dst):
    n_edges = src.shape[0]
    per_w = n_edges // _NW
    full = per_w // _CH
    rem = per_w - full * _CH
    groups = full // _RING
    assert groups * _RING == full
    mesh = plsc.VectorSubcoreMesh(core_axis_name="c", subcore_axis_name="s")

    @functools.partial(
        pl.kernel,
        mesh=mesh,
        out_type=jax.ShapeDtypeStruct((n_edges, H), jnp.float32),
        scratch_types=(
            [pltpu.VMEM((per_w,), jnp.int32)] * 2
            + [pltpu.VMEM((_CH, H), jnp.float32)] * (2 * _RING)
            + [pltpu.SemaphoreType.DMA] * (2 * _RING)
        ),
    )
    def k(u_hbm, v_hbm, src_hbm, dst_hbm, out_hbm, is_all, id_all, *rest):
        bufs_u = rest[0:_RING]
        bufs_v = rest[_RING:2 * _RING]
        sg = rest[2 * _RING:3 * _RING]
        sw = rest[3 * _RING:4 * _RING]
        wid = lax.axis_index("s") * 2 + lax.axis_index("c")
        w0 = wid * per_w

        pltpu.sync_copy(src_hbm.at[pl.ds(w0, per_w)], is_all)
        pltpu.sync_copy(dst_hbm.at[pl.ds(w0, per_w)], id_all)

        def issue_g(c, b):
            s = pl.ds(c * _CH, _CH)
            pltpu.async_copy(u_hbm.at[is_all.at[s]], bufs_u[b], sg[b])
            pltpu.async_copy(v_hbm.at[id_all.at[s]], bufs_v[b], sg[b])

        def wait_g(b):
            pltpu.make_async_copy(u_hbm.at[pl.ds(0, _CH)], bufs_u[b],
                                  sg[b]).wait()
            pltpu.make_async_copy(v_hbm.at[pl.ds(0, _CH)], bufs_v[b],
                                  sg[b]).wait()

        def add_uv(b, nrows):
            bu, bv = bufs_u[b], bufs_v[b]

            def row(r, carry):
                for cc in range(H // 16):
                    cs = pl.ds(cc * 16, 16)
                    bu[r, cs] = bu[r, cs] + bv[r, cs]
                return carry

            lax.fori_loop(0, nrows, row, 0)

        def issue_w(c, b):
            base = w0 + c * _CH
            pltpu.async_copy(bufs_u[b], out_hbm.at[pl.ds(base, _CH)], sw[b])

        def wait_w(b):
            pltpu.make_async_copy(bufs_u[b], out_hbm.at[pl.ds(0, _CH)],
                                  sw[b]).wait()

        def group(g, carry):
            for b in range(_RING):
                c = g * _RING + b
                pb = (b - 1) % _RING

                @pl.when(g > 0)
                def _():
                    wait_w(b)

                issue_g(c, b)
                if b == 0:
                    @pl.when(g > 0)
                    def _():
                        wait_g(pb)
                        add_uv(pb, _CH)
                        issue_w(c - 1, pb)
                else:
                    wait_g(pb)
                    add_uv(pb, _CH)
                    issue_w(c - 1, pb)
            return carry

        lax.fori_loop(0, groups, group, 0)
        lb = (full - 1) % _RING
        wait_g(lb)
        add_uv(lb, _CH)
        issue_w(full - 1, lb)
        for b in range(_RING):
            wait_w(b)
        if rem:
            base = full * _CH
            s = pl.ds(base, rem)
            d = pl.ds(0, rem)
            pltpu.async_copy(u_hbm.at[is_all.at[s]], bufs_u[0].at[d],
                             sg[0]).wait()
            pltpu.async_copy(v_hbm.at[id_all.at[s]], bufs_v[0].at[d],
                             sg[0]).wait()
            add_uv(0, rem)
            pltpu.sync_copy(bufs_u[0].at[d], out_hbm.at[pl.ds(w0 + base, rem)])

    return k(u, v, src, dst)


def _sc_scatter(parts, dst, zeros_nh, emit_ce=False):
    n = zeros_nh.shape[0]
    rows_per_tile = n // 16
    total = sum(arr.shape[0] for arr, _ in parts)
    mesh = plsc.VectorSubcoreMesh(core_axis_name="c", subcore_axis_name="s")

    meta = []
    for arr, off in parts:
        per_w = arr.shape[0] // _NW
        full = per_w // _CHS
        rem = per_w - full * _CHS
        assert full > 0 and full % _RING == 0
        meta.append((per_w, full, rem, off))
    rmax = max(max(m[2] for m in meta), 8)

    out_type = [jax.ShapeDtypeStruct((2, n, H), jnp.float32)]
    if emit_ce:
        out_type.append(jax.ShapeDtypeStruct((total, H), jnp.float32))

    @functools.partial(
        pl.kernel,
        mesh=mesh,
        out_type=tuple(out_type),
        scratch_types=(
            [pltpu.VMEM((_CHS,), jnp.int32)] * _RING
            + [pltpu.VMEM((_CHS, H), jnp.float32)] * _RING
            + [pltpu.VMEM((rmax,), jnp.int32),
               pltpu.VMEM((rmax, H), jnp.float32)]
            + [pltpu.VMEM_SHARED((n, H), jnp.float32)]
            + [pltpu.SemaphoreType.DMA] * (3 * _RING)
        ),
    )
    def k(*refs):
        np_ = len(parts)
        part_refs = refs[0:np_]
        dst_hbm = refs[np_]
        z_hbm = refs[np_ + 1]
        agg_hbm = refs[np_ + 2]
        pos = np_ + 3
        ce_hbm = refs[pos] if emit_ce else None
        pos += 1 if emit_ce else 0
        rest = refs[pos:]
        idx = rest[0:_RING]
        rows = rest[_RING:2 * _RING]
        idx_r = rest[2 * _RING]
        rows_r = rest[2 * _RING + 1]
        acc = rest[2 * _RING + 2]
        sl = rest[2 * _RING + 3:3 * _RING + 3]
        ss = rest[3 * _RING + 3:4 * _RING + 3]
        sc = rest[4 * _RING + 3:5 * _RING + 3]
        cid = lax.axis_index("c")
        sid = lax.axis_index("s")
        wid = sid * 2 + cid
        r0 = sid * rows_per_tile

        pltpu.sync_copy(z_hbm.at[pl.ds(r0, rows_per_tile)],
                        acc.at[pl.ds(r0, rows_per_tile)])
        plsc.subcore_barrier()

        for pi in range(np_):
            e_hbm = part_refs[pi]
            per_w, full, rem, off = meta[pi]
            w0l = wid * per_w
            groups = full // _RING

            def issue_l(c, b):
                bl = w0l + c * _CHS
                pltpu.async_copy(dst_hbm.at[pl.ds(off + bl, _CHS)], idx[b],
                                 sl[b])
                pltpu.async_copy(e_hbm.at[pl.ds(bl, _CHS)], rows[b], sl[b])

            def wait_l(b):
                pltpu.make_async_copy(dst_hbm.at[pl.ds(0, _CHS)], idx[b],
                                      sl[b]).wait()
                pltpu.make_async_copy(e_hbm.at[pl.ds(0, _CHS)], rows[b],
                                      sl[b]).wait()

            def issue_s(c, b):
                pltpu.async_copy(rows[b], acc.at[idx[b]], ss[b], add=True)
                if emit_ce:
                    base = off + w0l + c * _CHS
                    pltpu.async_copy(rows[b], ce_hbm.at[pl.ds(base, _CHS)],
                                     sc[b])

            def wait_s(b):
                pltpu.make_async_copy(rows[b], acc.at[pl.ds(0, _CHS)],
                                      ss[b]).wait()
                if emit_ce:
                    pltpu.make_async_copy(rows[b], ce_hbm.at[pl.ds(0, _CHS)],
                                          sc[b]).wait()

            def group(g, carry):
                for b in range(_RING):
                    c = g * _RING + b
                    pb = (b - 1) % _RING

                    @pl.when(g > 0)
                    def _():
                        wait_s(b)

                    issue_l(c, b)
                    if b == 0:
                        @pl.when(g > 0)
                        def _():
                            wait_l(pb)
                            issue_s(c - 1, pb)
                    else:
                        wait_l(pb)
                        issue_s(c - 1, pb)
                return carry

            lax.fori_loop(0, groups, group, 0)
            lb = (full - 1) % _RING
            wait_l(lb)
            issue_s(full - 1, lb)
            for b in range(_RING):
                wait_s(b)
            if rem:
                bl = w0l + full * _CHS
                rs = pl.ds(0, rem)
                pltpu.sync_copy(dst_hbm.at[pl.ds(off + bl, rem)],
                                idx_r.at[rs] if rem != rmax else idx_r)
                pltpu.sync_copy(e_hbm.at[pl.ds(bl, rem)],
                                rows_r.at[rs] if rem != rmax else rows_r)
                pltpu.sync_copy(rows_r.at[rs] if rem != rmax else rows_r,
                                acc.at[idx_r.at[rs] if rem != rmax else idx_r],
                                add=True)
                if emit_ce:
                    pltpu.sync_copy(
                        rows_r.at[rs] if rem != rmax else rows_r,
                        ce_hbm.at[pl.ds(off + bl, rem)])

        plsc.subcore_barrier()
        pltpu.sync_copy(acc.at[pl.ds(r0, rows_per_tile)],
                        agg_hbm.at[cid, pl.ds(r0, rows_per_tile)])

    out = k(*[arr for arr, _ in parts], dst, zeros_nh)
    return out if emit_ce else out[0]


def kernel(x, edge_attr, edge_index, params):
    src = edge_index[0]
    dst = edge_index[1]
    n = x.shape[0]
    ne = edge_attr.shape[0]
    el = ne // 2
    n_acc = ((n + 127) // 128) * 128
    zeros_nh = jnp.zeros((n_acc, H), dtype=jnp.float32)
    src_l, src_r = src[:el], src[el:]
    dst_l, dst_r = dst[:el], dst[el:]

    cx = x
    ce_l, ce_r = None, None
    ce_out = None
    for bi, p in enumerate(params):
        a = p['We1'][0:H]
        b = p['We1'][H:2 * H]
        c = p['We1'][2 * H:3 * H]
        w2 = p['We2']
        b1 = p['be1'].reshape(1, H)
        b2 = p['be2'].reshape(1, H)
        wa = p['Wn1'][0:H]
        wb = p['Wn1'][H:2 * H]
        n1 = p['bn1'].reshape(1, H)
        n2 = p['bn2'].reshape(1, H)
        last = bi == len(params) - 1

        u, v = _proj(cx, a, b)
        g_l = _sc_gather(u, v, src_l, dst_l)
        g_r = _sc_gather(u, v, src_r, dst_r)
        if bi == 0:
            e_l = _edge_mlp(g_l, edge_attr, 0, c, w2, b1, b2)
            e_r = _edge_mlp(g_r, edge_attr, el // _RB, c, w2, b1, b2)
        else:
            e_l = _edge_mlp(g_l, ce_l, 0, c, w2, b1, b2)
            e_r = _edge_mlp(g_r, ce_r, 0, c, w2, b1, b2)
        if last:
            agg, ce_out = _sc_scatter([(e_l, 0), (e_r, el)], dst, zeros_nh,
                                      emit_ce=True)
            aggs = [agg]
        else:
            agg_l = _sc_scatter([(e_l, 0)], dst, zeros_nh)
            agg_r = _sc_scatter([(e_r, el)], dst, zeros_nh)
            aggs = [agg_l, agg_r]
        cx = _node_mlp(cx, aggs, wa, wb, p['Wn2'], n1, n2)
        ce_l, ce_r = e_l, e_r

    return (cx, ce_out)


# speedup vs baseline: 4.6704x; 1.0493x over previous
import functools

import jax
import jax.numpy as jnp
from jax import lax
from jax.experimental import pallas as pl
from jax.experimental.pallas import tpu as pltpu
from jax.experimental.pallas import tpu_sc as plsc

H = 128
_RB = 1600
_NB = 1000
_CH = 128
_CHS = 104
_NW = 32
_RING = 3


def _proj_body(x_ref, a_ref, b_ref, u_ref, v_ref):
    x = x_ref[...]
    u_ref[...] = jnp.dot(x, a_ref[...], preferred_element_type=jnp.float32)
    v_ref[...] = jnp.dot(x, b_ref[...], preferred_element_type=jnp.float32)


def _proj(x, a, b):
    n = x.shape[0]
    return pl.pallas_call(
        _proj_body,
        grid=(n // _NB,),
        in_specs=[
            pl.BlockSpec((_NB, H), lambda i: (i, 0)),
            pl.BlockSpec((H, H), lambda i: (0, 0)),
            pl.BlockSpec((H, H), lambda i: (0, 0)),
        ],
        out_specs=[
            pl.BlockSpec((_NB, H), lambda i: (i, 0)),
            pl.BlockSpec((_NB, H), lambda i: (i, 0)),
        ],
        out_shape=[
            jax.ShapeDtypeStruct((n, H), jnp.float32),
            jax.ShapeDtypeStruct((n, H), jnp.float32),
        ],
    )(x, a, b)


def _edge_body(g_ref, e_ref, c_ref, w2_ref, b1_ref, b2_ref, o_ref):
    e = e_ref[...]
    pre = (g_ref[...] + b1_ref[...]
           + jnp.dot(e, c_ref[...], preferred_element_type=jnp.float32))
    eh = jnp.maximum(pre, 0.0)
    o_ref[...] = (jnp.dot(eh, w2_ref[...], preferred_element_type=jnp.float32)
                  + b2_ref[...] + e)


def _edge_mlp(g, e, e_off_blocks, c, w2, b1, b2):
    ne = g.shape[0]
    return pl.pallas_call(
        _edge_body,
        grid=(ne // _RB,),
        in_specs=[
            pl.BlockSpec((_RB, H), lambda i: (i, 0)),
            pl.BlockSpec((_RB, H), lambda i: (i + e_off_blocks, 0)),
            pl.BlockSpec((H, H), lambda i: (0, 0)),
            pl.BlockSpec((H, H), lambda i: (0, 0)),
            pl.BlockSpec((1, H), lambda i: (0, 0)),
            pl.BlockSpec((1, H), lambda i: (0, 0)),
        ],
        out_specs=pl.BlockSpec((_RB, H), lambda i: (i, 0)),
        out_shape=jax.ShapeDtypeStruct((ne, H), jnp.float32),
    )(g, e, c, w2, b1, b2)


def _node_mlp(x, aggs, wa, wb, w2, b1, b2):
    n = x.shape[0]
    na = len(aggs)

    def body(*refs):
        x_ref = refs[0]
        agg_refs = refs[1:1 + 2 * na]
        wa_ref, wb_ref, w2_ref, b1_ref, b2_ref, o_ref = refs[1 + 2 * na:]
        x = x_ref[...]
        a = agg_refs[0][0]
        for r in agg_refs[1:]:
            a = a + r[0]
        pre = (jnp.dot(x, wa_ref[...], preferred_element_type=jnp.float32)
               + jnp.dot(a, wb_ref[...], preferred_element_type=jnp.float32)
               + b1_ref[...])
        nh = jnp.maximum(pre, 0.0)
        o_ref[...] = (
            jnp.dot(nh, w2_ref[...], preferred_element_type=jnp.float32)
            + b2_ref[...] + x)

    agg_specs = []
    for _ in aggs:
        agg_specs.append(pl.BlockSpec((1, _NB, H), lambda i: (0, i, 0)))
        agg_specs.append(pl.BlockSpec((1, _NB, H), lambda i: (1, i, 0)))
    agg_args = [a for a in aggs for _ in range(2)]
    return pl.pallas_call(
        body,
        grid=(n // _NB,),
        in_specs=(
            [pl.BlockSpec((_NB, H), lambda i: (i, 0))]
            + agg_specs
            + [pl.BlockSpec((H, H), lambda i: (0, 0))] * 3
            + [pl.BlockSpec((1, H), lambda i: (0, 0))] * 2
        ),
        out_specs=pl.BlockSpec((_NB, H), lambda i: (i, 0)),
        out_shape=jax.ShapeDtypeStruct((n, H), jnp.float32),
    )(x, *agg_args, wa, wb, w2, b1, b2)


def _sc_gather(u, v, src, dst):
    n_edges = src.shape[0]
    per_w = n_edges // _NW
    full = per_w // _CH
    rem = per_w - full * _CH
    groups = full // _RING
    assert groups * _RING == full
    mesh = plsc.VectorSubcoreMesh(core_axis_name="c", subcore_axis_name="s")

    @functools.partial(
        pl.kernel,
        mesh=mesh,
        out_type=jax.ShapeDtypeStruct((n_edges, H), jnp.float32),
        scratch_types=(
            [pltpu.VMEM((per_w,), jnp.int32)] * 2
            + [pltpu.VMEM((_CH, H), jnp.float32)] * (2 * _RING)
            + [pltpu.SemaphoreType.DMA] * (2 * _RING)
        ),
    )
    def k(u_hbm, v_hbm, src_hbm, dst_hbm, out_hbm, is_all, id_all, *rest):
        bufs_u = rest[0:_RING]
        bufs_v = rest[_RING:2 * _RING]
        sg = rest[2 * _RING:3 * _RING]
        sw = rest[3 * _RING:4 * _RING]
        wid = lax.axis_index("s") * 2 + lax.axis_index("c")
        w0 = wid * per_w

        pltpu.sync_copy(src_hbm.at[pl.ds(w0, per_w)], is_all)
        pltpu.sync_copy(dst_hbm.at[pl.ds(w0, per_w)], id_all)

        def issue_g(c, b):
            s = pl.ds(c * _CH, _CH)
            pltpu.async_copy(u_hbm.at[is_all.at[s]], bufs_u[b], sg[b])
            pltpu.async_copy(v_hbm.at[id_all.at[s]], bufs_v[b], sg[b])

        def wait_g(b):
            pltpu.make_async_copy(u_hbm.at[pl.ds(0, _CH)], bufs_u[b],
                                  sg[b]).wait()
            pltpu.make_async_copy(v_hbm.at[pl.ds(0, _CH)], bufs_v[b],
                                  sg[b]).wait()

        def add_uv(b, nrows):
            bu, bv = bufs_u[b], bufs_v[b]

            def row(r, carry):
                for cc in range(H // 16):
                    cs = pl.ds(cc * 16, 16)
                    bu[r, cs] = bu[r, cs] + bv[r, cs]
                return carry

            lax.fori_loop(0, nrows, row, 0)

        def issue_w(c, b):
            base = w0 + c * _CH
            pltpu.async_copy(bufs_u[b], out_hbm.at[pl.ds(base, _CH)], sw[b])

        def wait_w(b):
            pltpu.make_async_copy(bufs_u[b], out_hbm.at[pl.ds(0, _CH)],
                                  sw[b]).wait()

        def group(g, carry):
            for b in range(_RING):
                c = g * _RING + b
                pb = (b - 1) % _RING

                @pl.when(g > 0)
                def _():
                    wait_w(b)

                issue_g(c, b)
                if b == 0:
                    @pl.when(g > 0)
                    def _():
                        wait_g(pb)
                        add_uv(pb, _CH)
                        issue_w(c - 1, pb)
                else:
                    wait_g(pb)
                    add_uv(pb, _CH)
                    issue_w(c - 1, pb)
            return carry

        lax.fori_loop(0, groups, group, 0)
        lb = (full - 1) % _RING
        wait_g(lb)
        add_uv(lb, _CH)
        issue_w(full - 1, lb)
        for b in range(_RING):
            wait_w(b)
        if rem:
            base = full * _CH
            s = pl.ds(base, rem)
            d = pl.ds(0, rem)
            pltpu.async_copy(u_hbm.at[is_all.at[s]], bufs_u[0].at[d],
                             sg[0]).wait()
            pltpu.async_copy(v_hbm.at[id_all.at[s]], bufs_v[0].at[d],
                             sg[0]).wait()
            add_uv(0, rem)
            pltpu.sync_copy(bufs_u[0].at[d], out_hbm.at[pl.ds(w0 + base, rem)])

    return k(u, v, src, dst)


def _sc_scatter(parts, dst, zeros_nh, emit_ce=False):
    n = zeros_nh.shape[0]
    rows_per_tile = n // 16
    total = sum(arr.shape[0] for arr, _ in parts)
    mesh = plsc.VectorSubcoreMesh(core_axis_name="c", subcore_axis_name="s")

    meta = []
    for arr, off in parts:
        per_w = arr.shape[0] // _NW
        full = per_w // _CHS
        rem = per_w - full * _CHS
        assert full > 0 and full % _RING == 0
        meta.append((per_w, full, rem, off))
    rmax = max(max(m[2] for m in meta), 8)

    out_type = [jax.ShapeDtypeStruct((2, n, H), jnp.float32)]
    if emit_ce:
        out_type.append(jax.ShapeDtypeStruct((total, H), jnp.float32))

    @functools.partial(
        pl.kernel,
        mesh=mesh,
        out_type=tuple(out_type),
        scratch_types=(
            [pltpu.VMEM((_CHS,), jnp.int32)] * _RING
            + [pltpu.VMEM((_CHS, H), jnp.float32)] * _RING
            + [pltpu.VMEM((rmax,), jnp.int32),
               pltpu.VMEM((rmax, H), jnp.float32)]
            + [pltpu.VMEM_SHARED((n, H), jnp.float32)]
            + [pltpu.SemaphoreType.DMA] * (3 * _RING)
        ),
    )
    def k(*refs):
        np_ = len(parts)
        part_refs = refs[0:np_]
        dst_hbm = refs[np_]
        z_hbm = refs[np_ + 1]
        agg_hbm = refs[np_ + 2]
        pos = np_ + 3
        ce_hbm = refs[pos] if emit_ce else None
        pos += 1 if emit_ce else 0
        rest = refs[pos:]
        idx = rest[0:_RING]
        rows = rest[_RING:2 * _RING]
        idx_r = rest[2 * _RING]
        rows_r = rest[2 * _RING + 1]
        acc = rest[2 * _RING + 2]
        sl = rest[2 * _RING + 3:3 * _RING + 3]
        ss = rest[3 * _RING + 3:4 * _RING + 3]
        sc = rest[4 * _RING + 3:5 * _RING + 3]
        cid = lax.axis_index("c")
        sid = lax.axis_index("s")
        wid = sid * 2 + cid
        r0 = sid * rows_per_tile

        pltpu.sync_copy(z_hbm.at[pl.ds(r0, rows_per_tile)],
                        acc.at[pl.ds(r0, rows_per_tile)])
        plsc.subcore_barrier()

        for pi in range(np_):
            e_hbm = part_refs[pi]
            per_w, full, rem, off = meta[pi]
            w0l = wid * per_w
            groups = full // _RING

            def issue_l(c, b):
                bl = w0l + c * _CHS
                pltpu.async_copy(dst_hbm.at[pl.ds(off + bl, _CHS)], idx[b],
                                 sl[b])
                pltpu.async_copy(e_hbm.at[pl.ds(bl, _CHS)], rows[b], sl[b])

            def wait_l(b):
                pltpu.make_async_copy(dst_hbm.at[pl.ds(0, _CHS)], idx[b],
                                      sl[b]).wait()
                pltpu.make_async_copy(e_hbm.at[pl.ds(0, _CHS)], rows[b],
                                      sl[b]).wait()

            def issue_s(c, b):
                pltpu.async_copy(rows[b], acc.at[idx[b]], ss[b], add=True)
                if emit_ce:
                    base = off + w0l + c * _CHS
                    pltpu.async_copy(rows[b], ce_hbm.at[pl.ds(base, _CHS)],
                                     sc[b])

            def wait_s(b):
                pltpu.make_async_copy(rows[b], acc.at[pl.ds(0, _CHS)],
                                      ss[b]).wait()
                if emit_ce:
                    pltpu.make_async_copy(rows[b], ce_hbm.at[pl.ds(0, _CHS)],
                                          sc[b]).wait()

            def group(g, carry):
                for b in range(_RING):
                    c = g * _RING + b
                    pb = (b - 1) % _RING

                    @pl.when(g > 0)
                    def _():
                        wait_s(b)

                    issue_l(c, b)
                    if b == 0:
                        @pl.when(g > 0)
                        def _():
                            wait_l(pb)
                            issue_s(c - 1, pb)
                    else:
                        wait_l(pb)
                        issue_s(c - 1, pb)
                return carry

            lax.fori_loop(0, groups, group, 0)
            lb = (full - 1) % _RING
            wait_l(lb)
            issue_s(full - 1, lb)
            for b in range(_RING):
                wait_s(b)
            if rem:
                bl = w0l + full * _CHS
                rs = pl.ds(0, rem)
                pltpu.sync_copy(dst_hbm.at[pl.ds(off + bl, rem)],
                                idx_r.at[rs] if rem != rmax else idx_r)
                pltpu.sync_copy(e_hbm.at[pl.ds(bl, rem)],
                                rows_r.at[rs] if rem != rmax else rows_r)
                pltpu.sync_copy(rows_r.at[rs] if rem != rmax else rows_r,
                                acc.at[idx_r.at[rs] if rem != rmax else idx_r],
                                add=True)
                if emit_ce:
                    pltpu.sync_copy(
                        rows_r.at[rs] if rem != rmax else rows_r,
                        ce_hbm.at[pl.ds(off + bl, rem)])

        plsc.subcore_barrier()
        pltpu.sync_copy(acc.at[pl.ds(r0, rows_per_tile)],
                        agg_hbm.at[cid, pl.ds(r0, rows_per_tile)])

    out = k(*[arr for arr, _ in parts], dst, zeros_nh)
    return out if emit_ce else out[0]


def kernel(x, edge_attr, edge_index, params):
    src = edge_index[0]
    dst = edge_index[1]
    n = x.shape[0]
    ne = edge_attr.shape[0]
    el = ne // 2
    n_acc = ((n + 127) // 128) * 128
    zeros_nh = jnp.zeros((n_acc, H), dtype=jnp.float32)
    src_l, src_r = src[:el], src[el:]
    dst_l, dst_r = dst[:el], dst[el:]

    cx = x
    ce_l, ce_r = None, None
    ce_out = None
    for bi, p in enumerate(params):
        a = p['We1'][0:H]
        b = p['We1'][H:2 * H]
        c = p['We1'][2 * H:3 * H]
        w2 = p['We2']
        b1 = p['be1'].reshape(1, H)
        b2 = p['be2'].reshape(1, H)
        wa = p['Wn1'][0:H]
        wb = p['Wn1'][H:2 * H]
        n1 = p['bn1'].reshape(1, H)
        n2 = p['bn2'].reshape(1, H)
        last = bi == len(params) - 1

        u, v = _proj(cx, a, b)
        g_l = _sc_gather(u, v, src_l, dst_l)
        g_r = _sc_gather(u, v, src_r, dst_r)
        if bi == 0:
            e_l = _edge_mlp(g_l, edge_attr, 0, c, w2, b1, b2)
            e_r = _edge_mlp(g_r, edge_attr, el // _RB, c, w2, b1, b2)
        else:
            e_l = _edge_mlp(g_l, ce_l, 0, c, w2, b1, b2)
            e_r = _edge_mlp(g_r, ce_r, 0, c, w2, b1, b2)
        if last:
            agg, ce_out = _sc_scatter([(e_l, 0), (e_r, el)], dst, zeros_nh,
                                      emit_ce=True)
            aggs = [agg]
        else:
            agg_l = _sc_scatter([(e_l, 0)], dst, zeros_nh)
            agg_r = _sc_scatter([(e_r, el)], dst, zeros_nh)
            aggs = [agg_l, agg_r]
        cx = _node_mlp(cx, aggs, wa, wb, p['Wn2'], n1, n2)
        ce_l, ce_r = e_l, e_r

    return (cx, ce_out)


# speedup vs baseline: 4.9194x; 1.0533x over previous
import functools

import jax
import jax.numpy as jnp
from jax import lax
from jax.experimental import pallas as pl
from jax.experimental.pallas import tpu as pltpu
from jax.experimental.pallas import tpu_sc as plsc

H = 128
_RB = 2000
_NB = 2000
_CH = 128
_CHS = 104
_NW = 32
_RING = 3


def _proj_body(x_ref, a_ref, b_ref, u_ref, v_ref):
    x = x_ref[...]
    u_ref[...] = jnp.dot(x, a_ref[...], preferred_element_type=jnp.float32)
    v_ref[...] = jnp.dot(x, b_ref[...], preferred_element_type=jnp.float32)


def _proj(x, a, b):
    n = x.shape[0]
    return pl.pallas_call(
        _proj_body,
        grid=(n // _NB,),
        in_specs=[
            pl.BlockSpec((_NB, H), lambda i: (i, 0)),
            pl.BlockSpec((H, H), lambda i: (0, 0)),
            pl.BlockSpec((H, H), lambda i: (0, 0)),
        ],
        out_specs=[
            pl.BlockSpec((_NB, H), lambda i: (i, 0)),
            pl.BlockSpec((_NB, H), lambda i: (i, 0)),
        ],
        out_shape=[
            jax.ShapeDtypeStruct((n, H), jnp.float32),
            jax.ShapeDtypeStruct((n, H), jnp.float32),
        ],
    )(x, a, b)


def _edge_body(g_ref, e_ref, c_ref, w2_ref, b1_ref, b2_ref, o_ref):
    e = e_ref[...]
    pre = (g_ref[...] + b1_ref[...]
           + jnp.dot(e, c_ref[...], preferred_element_type=jnp.float32))
    eh = jnp.maximum(pre, 0.0)
    o_ref[...] = (jnp.dot(eh, w2_ref[...], preferred_element_type=jnp.float32)
                  + b2_ref[...] + e)


def _edge_mlp(g, e, e_off_blocks, c, w2, b1, b2):
    ne = g.shape[0]
    return pl.pallas_call(
        _edge_body,
        grid=(ne // _RB,),
        in_specs=[
            pl.BlockSpec((_RB, H), lambda i: (i, 0)),
            pl.BlockSpec((_RB, H), lambda i: (i + e_off_blocks, 0)),
            pl.BlockSpec((H, H), lambda i: (0, 0)),
            pl.BlockSpec((H, H), lambda i: (0, 0)),
            pl.BlockSpec((1, H), lambda i: (0, 0)),
            pl.BlockSpec((1, H), lambda i: (0, 0)),
        ],
        out_specs=pl.BlockSpec((_RB, H), lambda i: (i, 0)),
        out_shape=jax.ShapeDtypeStruct((ne, H), jnp.float32),
    )(g, e, c, w2, b1, b2)


def _node_mlp(x, aggs, wa, wb, w2, b1, b2):
    n = x.shape[0]
    na = len(aggs)

    def body(*refs):
        x_ref = refs[0]
        agg_refs = refs[1:1 + 2 * na]
        wa_ref, wb_ref, w2_ref, b1_ref, b2_ref, o_ref = refs[1 + 2 * na:]
        x = x_ref[...]
        a = agg_refs[0][0]
        for r in agg_refs[1:]:
            a = a + r[0]
        pre = (jnp.dot(x, wa_ref[...], preferred_element_type=jnp.float32)
               + jnp.dot(a, wb_ref[...], preferred_element_type=jnp.float32)
               + b1_ref[...])
        nh = jnp.maximum(pre, 0.0)
        o_ref[...] = (
            jnp.dot(nh, w2_ref[...], preferred_element_type=jnp.float32)
            + b2_ref[...] + x)

    agg_specs = []
    for _ in aggs:
        agg_specs.append(pl.BlockSpec((1, _NB, H), lambda i: (0, i, 0)))
        agg_specs.append(pl.BlockSpec((1, _NB, H), lambda i: (1, i, 0)))
    agg_args = [a for a in aggs for _ in range(2)]
    return pl.pallas_call(
        body,
        grid=(n // _NB,),
        in_specs=(
            [pl.BlockSpec((_NB, H), lambda i: (i, 0))]
            + agg_specs
            + [pl.BlockSpec((H, H), lambda i: (0, 0))] * 3
            + [pl.BlockSpec((1, H), lambda i: (0, 0))] * 2
        ),
        out_specs=pl.BlockSpec((_NB, H), lambda i: (i, 0)),
        out_shape=jax.ShapeDtypeStruct((n, H), jnp.float32),
    )(x, *agg_args, wa, wb, w2, b1, b2)


def _sc_gather(u, v, src, dst):
    n_edges = src.shape[0]
    per_w = n_edges // _NW
    full = per_w // _CH
    rem = per_w - full * _CH
    groups = full // _RING
    assert groups * _RING == full
    mesh = plsc.VectorSubcoreMesh(core_axis_name="c", subcore_axis_name="s")

    @functools.partial(
        pl.kernel,
        mesh=mesh,
        out_type=jax.ShapeDtypeStruct((n_edges, H), jnp.float32),
        scratch_types=(
            [pltpu.VMEM((per_w,), jnp.int32)] * 2
            + [pltpu.VMEM((_CH, H), jnp.float32)] * (2 * _RING)
            + [pltpu.SemaphoreType.DMA] * (2 * _RING)
        ),
    )
    def k(u_hbm, v_hbm, src_hbm, dst_hbm, out_hbm, is_all, id_all, *rest):
        bufs_u = rest[0:_RING]
        bufs_v = rest[_RING:2 * _RING]
        sg = rest[2 * _RING:3 * _RING]
        sw = rest[3 * _RING:4 * _RING]
        wid = lax.axis_index("s") * 2 + lax.axis_index("c")
        w0 = wid * per_w

        pltpu.sync_copy(src_hbm.at[pl.ds(w0, per_w)], is_all)
        pltpu.sync_copy(dst_hbm.at[pl.ds(w0, per_w)], id_all)

        def issue_g(c, b):
            s = pl.ds(c * _CH, _CH)
            pltpu.async_copy(u_hbm.at[is_all.at[s]], bufs_u[b], sg[b])
            pltpu.async_copy(v_hbm.at[id_all.at[s]], bufs_v[b], sg[b])

        def wait_g(b):
            pltpu.make_async_copy(u_hbm.at[pl.ds(0, _CH)], bufs_u[b],
                                  sg[b]).wait()
            pltpu.make_async_copy(v_hbm.at[pl.ds(0, _CH)], bufs_v[b],
                                  sg[b]).wait()

        def add_uv(b, nrows):
            bu, bv = bufs_u[b], bufs_v[b]

            def row(r, carry):
                for cc in range(H // 16):
                    cs = pl.ds(cc * 16, 16)
                    bu[r, cs] = bu[r, cs] + bv[r, cs]
                return carry

            lax.fori_loop(0, nrows, row, 0)

        def issue_w(c, b):
            base = w0 + c * _CH
            pltpu.async_copy(bufs_u[b], out_hbm.at[pl.ds(base, _CH)], sw[b])

        def wait_w(b):
            pltpu.make_async_copy(bufs_u[b], out_hbm.at[pl.ds(0, _CH)],
                                  sw[b]).wait()

        def group(g, carry):
            for b in range(_RING):
                c = g * _RING + b
                pb = (b - 1) % _RING

                @pl.when(g > 0)
                def _():
                    wait_w(b)

                issue_g(c, b)
                if b == 0:
                    @pl.when(g > 0)
                    def _():
                        wait_g(pb)
                        add_uv(pb, _CH)
                        issue_w(c - 1, pb)
                else:
                    wait_g(pb)
                    add_uv(pb, _CH)
                    issue_w(c - 1, pb)
            return carry

        lax.fori_loop(0, groups, group, 0)
        lb = (full - 1) % _RING
        wait_g(lb)
        add_uv(lb, _CH)
        issue_w(full - 1, lb)
        for b in range(_RING):
            wait_w(b)
        if rem:
            base = full * _CH
            s = pl.ds(base, rem)
            d = pl.ds(0, rem)
            pltpu.async_copy(u_hbm.at[is_all.at[s]], bufs_u[0].at[d],
                             sg[0]).wait()
            pltpu.async_copy(v_hbm.at[id_all.at[s]], bufs_v[0].at[d],
                             sg[0]).wait()
            add_uv(0, rem)
            pltpu.sync_copy(bufs_u[0].at[d], out_hbm.at[pl.ds(w0 + base, rem)])

    return k(u, v, src, dst)


def _sc_scatter(parts, dst, zeros_nh, emit_ce=False):
    n = zeros_nh.shape[0]
    rows_per_tile = n // 16
    total = sum(arr.shape[0] for arr, _ in parts)
    mesh = plsc.VectorSubcoreMesh(core_axis_name="c", subcore_axis_name="s")

    meta = []
    for arr, off in parts:
        per_w = arr.shape[0] // _NW
        full = per_w // _CHS
        rem = per_w - full * _CHS
        assert full > 0 and full % _RING == 0
        meta.append((per_w, full, rem, off))
    rmax = max(max(m[2] for m in meta), 8)

    out_type = [jax.ShapeDtypeStruct((2, n, H), jnp.float32)]
    if emit_ce:
        out_type.append(jax.ShapeDtypeStruct((total, H), jnp.float32))

    @functools.partial(
        pl.kernel,
        mesh=mesh,
        out_type=tuple(out_type),
        scratch_types=(
            [pltpu.VMEM((_CHS,), jnp.int32)] * _RING
            + [pltpu.VMEM((_CHS, H), jnp.float32)] * _RING
            + [pltpu.VMEM((rmax,), jnp.int32),
               pltpu.VMEM((rmax, H), jnp.float32)]
            + [pltpu.VMEM_SHARED((n, H), jnp.float32)]
            + [pltpu.SemaphoreType.DMA] * (3 * _RING)
        ),
    )
    def k(*refs):
        np_ = len(parts)
        part_refs = refs[0:np_]
        dst_hbm = refs[np_]
        z_hbm = refs[np_ + 1]
        agg_hbm = refs[np_ + 2]
        pos = np_ + 3
        ce_hbm = refs[pos] if emit_ce else None
        pos += 1 if emit_ce else 0
        rest = refs[pos:]
        idx = rest[0:_RING]
        rows = rest[_RING:2 * _RING]
        idx_r = rest[2 * _RING]
        rows_r = rest[2 * _RING + 1]
        acc = rest[2 * _RING + 2]
        sl = rest[2 * _RING + 3:3 * _RING + 3]
        ss = rest[3 * _RING + 3:4 * _RING + 3]
        sc = rest[4 * _RING + 3:5 * _RING + 3]
        cid = lax.axis_index("c")
        sid = lax.axis_index("s")
        wid = sid * 2 + cid
        r0 = sid * rows_per_tile

        pltpu.sync_copy(z_hbm.at[pl.ds(r0, rows_per_tile)],
                        acc.at[pl.ds(r0, rows_per_tile)])
        plsc.subcore_barrier()

        for pi in range(np_):
            e_hbm = part_refs[pi]
            per_w, full, rem, off = meta[pi]
            w0l = wid * per_w
            groups = full // _RING

            def issue_l(c, b):
                bl = w0l + c * _CHS
                pltpu.async_copy(dst_hbm.at[pl.ds(off + bl, _CHS)], idx[b],
                                 sl[b])
                pltpu.async_copy(e_hbm.at[pl.ds(bl, _CHS)], rows[b], sl[b])

            def wait_l(b):
                pltpu.make_async_copy(dst_hbm.at[pl.ds(0, _CHS)], idx[b],
                                      sl[b]).wait()
                pltpu.make_async_copy(e_hbm.at[pl.ds(0, _CHS)], rows[b],
                                      sl[b]).wait()

            def issue_s(c, b):
                pltpu.async_copy(rows[b], acc.at[idx[b]], ss[b], add=True)
                if emit_ce:
                    base = off + w0l + c * _CHS
                    pltpu.async_copy(rows[b], ce_hbm.at[pl.ds(base, _CHS)],
                                     sc[b])

            def wait_s(b):
                pltpu.make_async_copy(rows[b], acc.at[pl.ds(0, _CHS)],
                                      ss[b]).wait()
                if emit_ce:
                    pltpu.make_async_copy(rows[b], ce_hbm.at[pl.ds(0, _CHS)],
                                          sc[b]).wait()

            def group(g, carry):
                for b in range(_RING):
                    c = g * _RING + b
                    pb = (b - 1) % _RING

                    @pl.when(g > 0)
                    def _():
                        wait_s(b)

                    issue_l(c, b)
                    if b == 0:
                        @pl.when(g > 0)
                        def _():
                            wait_l(pb)
                            issue_s(c - 1, pb)
                    else:
                        wait_l(pb)
                        issue_s(c - 1, pb)
                return carry

            lax.fori_loop(0, groups, group, 0)
            lb = (full - 1) % _RING
            wait_l(lb)
            issue_s(full - 1, lb)
            for b in range(_RING):
                wait_s(b)
            if rem:
                bl = w0l + full * _CHS
                rs = pl.ds(0, rem)
                pltpu.sync_copy(dst_hbm.at[pl.ds(off + bl, rem)],
                                idx_r.at[rs] if rem != rmax else idx_r)
                pltpu.sync_copy(e_hbm.at[pl.ds(bl, rem)],
                                rows_r.at[rs] if rem != rmax else rows_r)
                pltpu.sync_copy(rows_r.at[rs] if rem != rmax else rows_r,
                                acc.at[idx_r.at[rs] if rem != rmax else idx_r],
                                add=True)
                if emit_ce:
                    pltpu.sync_copy(
                        rows_r.at[rs] if rem != rmax else rows_r,
                        ce_hbm.at[pl.ds(off + bl, rem)])

        plsc.subcore_barrier()
        pltpu.sync_copy(acc.at[pl.ds(r0, rows_per_tile)],
                        agg_hbm.at[cid, pl.ds(r0, rows_per_tile)])

    out = k(*[arr for arr, _ in parts], dst, zeros_nh)
    return out if emit_ce else out[0]


def kernel(x, edge_attr, edge_index, params):
    src = edge_index[0]
    dst = edge_index[1]
    n = x.shape[0]
    ne = edge_attr.shape[0]
    el = ne // 2
    n_acc = ((n + 127) // 128) * 128
    zeros_nh = jnp.zeros((n_acc, H), dtype=jnp.float32)
    src_l, src_r = src[:el], src[el:]
    dst_l, dst_r = dst[:el], dst[el:]

    cx = x
    ce_l, ce_r = None, None
    ce_out = None
    for bi, p in enumerate(params):
        a = p['We1'][0:H]
        b = p['We1'][H:2 * H]
        c = p['We1'][2 * H:3 * H]
        w2 = p['We2']
        b1 = p['be1'].reshape(1, H)
        b2 = p['be2'].reshape(1, H)
        wa = p['Wn1'][0:H]
        wb = p['Wn1'][H:2 * H]
        n1 = p['bn1'].reshape(1, H)
        n2 = p['bn2'].reshape(1, H)
        last = bi == len(params) - 1

        u, v = _proj(cx, a, b)
        g_l = _sc_gather(u, v, src_l, dst_l)
        g_r = _sc_gather(u, v, src_r, dst_r)
        if bi == 0:
            e_l = _edge_mlp(g_l, edge_attr, 0, c, w2, b1, b2)
            e_r = _edge_mlp(g_r, edge_attr, el // _RB, c, w2, b1, b2)
        else:
            e_l = _edge_mlp(g_l, ce_l, 0, c, w2, b1, b2)
            e_r = _edge_mlp(g_r, ce_r, 0, c, w2, b1, b2)
        if last:
            agg, ce_out = _sc_scatter([(e_l, 0), (e_r, el)], dst, zeros_nh,
                                      emit_ce=True)
            aggs = [agg]
        else:
            agg_l = _sc_scatter([(e_l, 0)], dst, zeros_nh)
            agg_r = _sc_scatter([(e_r, el)], dst, zeros_nh)
            aggs = [agg_l, agg_r]
        cx = _node_mlp(cx, aggs, wa, wb, p['Wn2'], n1, n2)
        ce_l, ce_r = e_l, e_r

    return (cx, ce_out)


# speedup vs baseline: 5.1512x; 1.0471x over previous
import functools

import jax
import jax.numpy as jnp
from jax import lax
from jax.experimental import pallas as pl
from jax.experimental.pallas import tpu as pltpu
from jax.experimental.pallas import tpu_sc as plsc

H = 128
_RB = 4000
_NB = 5000
_CH = 128
_CHS = 104
_NW = 32
_RING = 3


def _proj_body(x_ref, a_ref, b_ref, u_ref, v_ref):
    x = x_ref[...]
    u_ref[...] = jnp.dot(x, a_ref[...], preferred_element_type=jnp.float32)
    v_ref[...] = jnp.dot(x, b_ref[...], preferred_element_type=jnp.float32)


def _proj(x, a, b):
    n = x.shape[0]
    return pl.pallas_call(
        _proj_body,
        grid=(n // _NB,),
        in_specs=[
            pl.BlockSpec((_NB, H), lambda i: (i, 0)),
            pl.BlockSpec((H, H), lambda i: (0, 0)),
            pl.BlockSpec((H, H), lambda i: (0, 0)),
        ],
        out_specs=[
            pl.BlockSpec((_NB, H), lambda i: (i, 0)),
            pl.BlockSpec((_NB, H), lambda i: (i, 0)),
        ],
        out_shape=[
            jax.ShapeDtypeStruct((n, H), jnp.float32),
            jax.ShapeDtypeStruct((n, H), jnp.float32),
        ],
    )(x, a, b)


def _edge_body(g_ref, e_ref, c_ref, w2_ref, b1_ref, b2_ref, o_ref):
    e = e_ref[...]
    pre = (g_ref[...] + b1_ref[...]
           + jnp.dot(e, c_ref[...], preferred_element_type=jnp.float32))
    eh = jnp.maximum(pre, 0.0)
    o_ref[...] = (jnp.dot(eh, w2_ref[...], preferred_element_type=jnp.float32)
                  + b2_ref[...] + e)


def _edge_mlp(g, e, e_off_blocks, c, w2, b1, b2):
    ne = g.shape[0]
    return pl.pallas_call(
        _edge_body,
        grid=(ne // _RB,),
        in_specs=[
            pl.BlockSpec((_RB, H), lambda i: (i, 0)),
            pl.BlockSpec((_RB, H), lambda i: (i + e_off_blocks, 0)),
            pl.BlockSpec((H, H), lambda i: (0, 0)),
            pl.BlockSpec((H, H), lambda i: (0, 0)),
            pl.BlockSpec((1, H), lambda i: (0, 0)),
            pl.BlockSpec((1, H), lambda i: (0, 0)),
        ],
        out_specs=pl.BlockSpec((_RB, H), lambda i: (i, 0)),
        out_shape=jax.ShapeDtypeStruct((ne, H), jnp.float32),
    )(g, e, c, w2, b1, b2)


def _node_mlp(x, aggs, wa, wb, w2, b1, b2):
    n = x.shape[0]
    na = len(aggs)

    def body(*refs):
        x_ref = refs[0]
        agg_refs = refs[1:1 + 2 * na]
        wa_ref, wb_ref, w2_ref, b1_ref, b2_ref, o_ref = refs[1 + 2 * na:]
        x = x_ref[...]
        a = agg_refs[0][0]
        for r in agg_refs[1:]:
            a = a + r[0]
        pre = (jnp.dot(x, wa_ref[...], preferred_element_type=jnp.float32)
               + jnp.dot(a, wb_ref[...], preferred_element_type=jnp.float32)
               + b1_ref[...])
        nh = jnp.maximum(pre, 0.0)
        o_ref[...] = (
            jnp.dot(nh, w2_ref[...], preferred_element_type=jnp.float32)
            + b2_ref[...] + x)

    agg_specs = []
    for _ in aggs:
        agg_specs.append(pl.BlockSpec((1, _NB, H), lambda i: (0, i, 0)))
        agg_specs.append(pl.BlockSpec((1, _NB, H), lambda i: (1, i, 0)))
    agg_args = [a for a in aggs for _ in range(2)]
    return pl.pallas_call(
        body,
        grid=(n // _NB,),
        in_specs=(
            [pl.BlockSpec((_NB, H), lambda i: (i, 0))]
            + agg_specs
            + [pl.BlockSpec((H, H), lambda i: (0, 0))] * 3
            + [pl.BlockSpec((1, H), lambda i: (0, 0))] * 2
        ),
        out_specs=pl.BlockSpec((_NB, H), lambda i: (i, 0)),
        out_shape=jax.ShapeDtypeStruct((n, H), jnp.float32),
    )(x, *agg_args, wa, wb, w2, b1, b2)


def _sc_gather(u, v, src, dst):
    n_edges = src.shape[0]
    per_w = n_edges // _NW
    full = per_w // _CH
    rem = per_w - full * _CH
    groups = full // _RING
    assert groups * _RING == full
    mesh = plsc.VectorSubcoreMesh(core_axis_name="c", subcore_axis_name="s")

    @functools.partial(
        pl.kernel,
        mesh=mesh,
        out_type=jax.ShapeDtypeStruct((n_edges, H), jnp.float32),
        scratch_types=(
            [pltpu.VMEM((per_w,), jnp.int32)] * 2
            + [pltpu.VMEM((_CH, H), jnp.float32)] * (2 * _RING)
            + [pltpu.SemaphoreType.DMA] * (2 * _RING)
        ),
    )
    def k(u_hbm, v_hbm, src_hbm, dst_hbm, out_hbm, is_all, id_all, *rest):
        bufs_u = rest[0:_RING]
        bufs_v = rest[_RING:2 * _RING]
        sg = rest[2 * _RING:3 * _RING]
        sw = rest[3 * _RING:4 * _RING]
        wid = lax.axis_index("s") * 2 + lax.axis_index("c")
        w0 = wid * per_w

        pltpu.sync_copy(src_hbm.at[pl.ds(w0, per_w)], is_all)
        pltpu.sync_copy(dst_hbm.at[pl.ds(w0, per_w)], id_all)

        def issue_g(c, b):
            s = pl.ds(c * _CH, _CH)
            pltpu.async_copy(u_hbm.at[is_all.at[s]], bufs_u[b], sg[b])
            pltpu.async_copy(v_hbm.at[id_all.at[s]], bufs_v[b], sg[b])

        def wait_g(b):
            pltpu.make_async_copy(u_hbm.at[pl.ds(0, _CH)], bufs_u[b],
                                  sg[b]).wait()
            pltpu.make_async_copy(v_hbm.at[pl.ds(0, _CH)], bufs_v[b],
                                  sg[b]).wait()

        def add_uv(b, nrows):
            bu, bv = bufs_u[b], bufs_v[b]

            def row(r, carry):
                for cc in range(H // 16):
                    cs = pl.ds(cc * 16, 16)
                    bu[r, cs] = bu[r, cs] + bv[r, cs]
                return carry

            lax.fori_loop(0, nrows, row, 0)

        def issue_w(c, b):
            base = w0 + c * _CH
            pltpu.async_copy(bufs_u[b], out_hbm.at[pl.ds(base, _CH)], sw[b])

        def wait_w(b):
            pltpu.make_async_copy(bufs_u[b], out_hbm.at[pl.ds(0, _CH)],
                                  sw[b]).wait()

        def group(g, carry):
            for b in range(_RING):
                c = g * _RING + b
                pb = (b - 1) % _RING

                @pl.when(g > 0)
                def _():
                    wait_w(b)

                issue_g(c, b)
                if b == 0:
                    @pl.when(g > 0)
                    def _():
                        wait_g(pb)
                        add_uv(pb, _CH)
                        issue_w(c - 1, pb)
                else:
                    wait_g(pb)
                    add_uv(pb, _CH)
                    issue_w(c - 1, pb)
            return carry

        lax.fori_loop(0, groups, group, 0)
        lb = (full - 1) % _RING
        wait_g(lb)
        add_uv(lb, _CH)
        issue_w(full - 1, lb)
        for b in range(_RING):
            wait_w(b)
        if rem:
            base = full * _CH
            s = pl.ds(base, rem)
            d = pl.ds(0, rem)
            pltpu.async_copy(u_hbm.at[is_all.at[s]], bufs_u[0].at[d],
                             sg[0]).wait()
            pltpu.async_copy(v_hbm.at[id_all.at[s]], bufs_v[0].at[d],
                             sg[0]).wait()
            add_uv(0, rem)
            pltpu.sync_copy(bufs_u[0].at[d], out_hbm.at[pl.ds(w0 + base, rem)])

    return k(u, v, src, dst)


def _sc_scatter(parts, dst, zeros_nh, emit_ce=False):
    n = zeros_nh.shape[0]
    rows_per_tile = n // 16
    total = sum(arr.shape[0] for arr, _ in parts)
    mesh = plsc.VectorSubcoreMesh(core_axis_name="c", subcore_axis_name="s")

    meta = []
    for arr, off in parts:
        per_w = arr.shape[0] // _NW
        full = per_w // _CHS
        rem = per_w - full * _CHS
        assert full > 0 and full % _RING == 0
        meta.append((per_w, full, rem, off))
    rmax = max(max(m[2] for m in meta), 8)

    out_type = [jax.ShapeDtypeStruct((2, n, H), jnp.float32)]
    if emit_ce:
        out_type.append(jax.ShapeDtypeStruct((total, H), jnp.float32))

    @functools.partial(
        pl.kernel,
        mesh=mesh,
        out_type=tuple(out_type),
        scratch_types=(
            [pltpu.VMEM((_CHS,), jnp.int32)] * _RING
            + [pltpu.VMEM((_CHS, H), jnp.float32)] * _RING
            + [pltpu.VMEM((rmax,), jnp.int32),
               pltpu.VMEM((rmax, H), jnp.float32)]
            + [pltpu.VMEM_SHARED((n, H), jnp.float32)]
            + [pltpu.SemaphoreType.DMA] * (3 * _RING)
        ),
    )
    def k(*refs):
        np_ = len(parts)
        part_refs = refs[0:np_]
        dst_hbm = refs[np_]
        z_hbm = refs[np_ + 1]
        agg_hbm = refs[np_ + 2]
        pos = np_ + 3
        ce_hbm = refs[pos] if emit_ce else None
        pos += 1 if emit_ce else 0
        rest = refs[pos:]
        idx = rest[0:_RING]
        rows = rest[_RING:2 * _RING]
        idx_r = rest[2 * _RING]
        rows_r = rest[2 * _RING + 1]
        acc = rest[2 * _RING + 2]
        sl = rest[2 * _RING + 3:3 * _RING + 3]
        ss = rest[3 * _RING + 3:4 * _RING + 3]
        sc = rest[4 * _RING + 3:5 * _RING + 3]
        cid = lax.axis_index("c")
        sid = lax.axis_index("s")
        wid = sid * 2 + cid
        r0 = sid * rows_per_tile

        pltpu.sync_copy(z_hbm.at[pl.ds(r0, rows_per_tile)],
                        acc.at[pl.ds(r0, rows_per_tile)])
        plsc.subcore_barrier()

        for pi in range(np_):
            e_hbm = part_refs[pi]
            per_w, full, rem, off = meta[pi]
            w0l = wid * per_w
            groups = full // _RING

            def issue_l(c, b):
                bl = w0l + c * _CHS
                pltpu.async_copy(dst_hbm.at[pl.ds(off + bl, _CHS)], idx[b],
                                 sl[b])
                pltpu.async_copy(e_hbm.at[pl.ds(bl, _CHS)], rows[b], sl[b])

            def wait_l(b):
                pltpu.make_async_copy(dst_hbm.at[pl.ds(0, _CHS)], idx[b],
                                      sl[b]).wait()
                pltpu.make_async_copy(e_hbm.at[pl.ds(0, _CHS)], rows[b],
                                      sl[b]).wait()

            def issue_s(c, b):
                pltpu.async_copy(rows[b], acc.at[idx[b]], ss[b], add=True)
                if emit_ce:
                    base = off + w0l + c * _CHS
                    pltpu.async_copy(rows[b], ce_hbm.at[pl.ds(base, _CHS)],
                                     sc[b])

            def wait_s(b):
                pltpu.make_async_copy(rows[b], acc.at[pl.ds(0, _CHS)],
                                      ss[b]).wait()
                if emit_ce:
                    pltpu.make_async_copy(rows[b], ce_hbm.at[pl.ds(0, _CHS)],
                                          sc[b]).wait()

            def group(g, carry):
                for b in range(_RING):
                    c = g * _RING + b
                    pb = (b - 1) % _RING

                    @pl.when(g > 0)
                    def _():
                        wait_s(b)

                    issue_l(c, b)
                    if b == 0:
                        @pl.when(g > 0)
                        def _():
                            wait_l(pb)
                            issue_s(c - 1, pb)
                    else:
                        wait_l(pb)
                        issue_s(c - 1, pb)
                return carry

            lax.fori_loop(0, groups, group, 0)
            lb = (full - 1) % _RING
            wait_l(lb)
            issue_s(full - 1, lb)
            for b in range(_RING):
                wait_s(b)
            if rem:
                bl = w0l + full * _CHS
                rs = pl.ds(0, rem)
                pltpu.sync_copy(dst_hbm.at[pl.ds(off + bl, rem)],
                                idx_r.at[rs] if rem != rmax else idx_r)
                pltpu.sync_copy(e_hbm.at[pl.ds(bl, rem)],
                                rows_r.at[rs] if rem != rmax else rows_r)
                pltpu.sync_copy(rows_r.at[rs] if rem != rmax else rows_r,
                                acc.at[idx_r.at[rs] if rem != rmax else idx_r],
                                add=True)
                if emit_ce:
                    pltpu.sync_copy(
                        rows_r.at[rs] if rem != rmax else rows_r,
                        ce_hbm.at[pl.ds(off + bl, rem)])

        plsc.subcore_barrier()
        pltpu.sync_copy(acc.at[pl.ds(r0, rows_per_tile)],
                        agg_hbm.at[cid, pl.ds(r0, rows_per_tile)])

    out = k(*[arr for arr, _ in parts], dst, zeros_nh)
    return out if emit_ce else out[0]


def kernel(x, edge_attr, edge_index, params):
    src = edge_index[0]
    dst = edge_index[1]
    n = x.shape[0]
    ne = edge_attr.shape[0]
    el = ne // 2
    n_acc = ((n + 127) // 128) * 128
    zeros_nh = jnp.zeros((n_acc, H), dtype=jnp.float32)
    src_l, src_r = src[:el], src[el:]
    dst_l, dst_r = dst[:el], dst[el:]

    cx = x
    ce_l, ce_r = None, None
    ce_out = None
    for bi, p in enumerate(params):
        a = p['We1'][0:H]
        b = p['We1'][H:2 * H]
        c = p['We1'][2 * H:3 * H]
        w2 = p['We2']
        b1 = p['be1'].reshape(1, H)
        b2 = p['be2'].reshape(1, H)
        wa = p['Wn1'][0:H]
        wb = p['Wn1'][H:2 * H]
        n1 = p['bn1'].reshape(1, H)
        n2 = p['bn2'].reshape(1, H)
        last = bi == len(params) - 1

        u, v = _proj(cx, a, b)
        g_l = _sc_gather(u, v, src_l, dst_l)
        g_r = _sc_gather(u, v, src_r, dst_r)
        if bi == 0:
            e_l = _edge_mlp(g_l, edge_attr, 0, c, w2, b1, b2)
            e_r = _edge_mlp(g_r, edge_attr, el // _RB, c, w2, b1, b2)
        else:
            e_l = _edge_mlp(g_l, ce_l, 0, c, w2, b1, b2)
            e_r = _edge_mlp(g_r, ce_r, 0, c, w2, b1, b2)
        if last:
            agg, ce_out = _sc_scatter([(e_l, 0), (e_r, el)], dst, zeros_nh,
                                      emit_ce=True)
            aggs = [agg]
        else:
            agg_l = _sc_scatter([(e_l, 0)], dst, zeros_nh)
            agg_r = _sc_scatter([(e_r, el)], dst, zeros_nh)
            aggs = [agg_l, agg_r]
        cx = _node_mlp(cx, aggs, wa, wb, p['Wn2'], n1, n2)
        ce_l, ce_r = e_l, e_r

    return (cx, ce_out)


# speedup vs baseline: 5.1976x; 1.0090x over previous
import functools

import jax
import jax.numpy as jnp
from jax import lax
from jax.experimental import pallas as pl
from jax.experimental.pallas import tpu as pltpu
from jax.experimental.pallas import tpu_sc as plsc

H = 128
_RB = 8000
_NB = 5000
_CH = 128
_CHS = 104
_NW = 32
_RING = 3


def _proj_body(x_ref, a_ref, b_ref, u_ref, v_ref):
    x = x_ref[...]
    u_ref[...] = jnp.dot(x, a_ref[...], preferred_element_type=jnp.float32)
    v_ref[...] = jnp.dot(x, b_ref[...], preferred_element_type=jnp.float32)


def _proj(x, a, b):
    n = x.shape[0]
    return pl.pallas_call(
        _proj_body,
        grid=(n // _NB,),
        in_specs=[
            pl.BlockSpec((_NB, H), lambda i: (i, 0)),
            pl.BlockSpec((H, H), lambda i: (0, 0)),
            pl.BlockSpec((H, H), lambda i: (0, 0)),
        ],
        out_specs=[
            pl.BlockSpec((_NB, H), lambda i: (i, 0)),
            pl.BlockSpec((_NB, H), lambda i: (i, 0)),
        ],
        out_shape=[
            jax.ShapeDtypeStruct((n, H), jnp.float32),
            jax.ShapeDtypeStruct((n, H), jnp.float32),
        ],
    )(x, a, b)


def _edge_body(g_ref, e_ref, c_ref, w2_ref, b1_ref, b2_ref, o_ref):
    e = e_ref[...]
    pre = (g_ref[...] + b1_ref[...]
           + jnp.dot(e, c_ref[...], preferred_element_type=jnp.float32))
    eh = jnp.maximum(pre, 0.0)
    o_ref[...] = (jnp.dot(eh, w2_ref[...], preferred_element_type=jnp.float32)
                  + b2_ref[...] + e)


def _edge_mlp(g, e, e_off_blocks, c, w2, b1, b2):
    ne = g.shape[0]
    return pl.pallas_call(
        _edge_body,
        grid=(ne // _RB,),
        in_specs=[
            pl.BlockSpec((_RB, H), lambda i: (i, 0)),
            pl.BlockSpec((_RB, H), lambda i: (i + e_off_blocks, 0)),
            pl.BlockSpec((H, H), lambda i: (0, 0)),
            pl.BlockSpec((H, H), lambda i: (0, 0)),
            pl.BlockSpec((1, H), lambda i: (0, 0)),
            pl.BlockSpec((1, H), lambda i: (0, 0)),
        ],
        out_specs=pl.BlockSpec((_RB, H), lambda i: (i, 0)),
        out_shape=jax.ShapeDtypeStruct((ne, H), jnp.float32),
    )(g, e, c, w2, b1, b2)


def _node_mlp(x, aggs, wa, wb, w2, b1, b2):
    n = x.shape[0]
    na = len(aggs)

    def body(*refs):
        x_ref = refs[0]
        agg_refs = refs[1:1 + 2 * na]
        wa_ref, wb_ref, w2_ref, b1_ref, b2_ref, o_ref = refs[1 + 2 * na:]
        x = x_ref[...]
        a = agg_refs[0][0]
        for r in agg_refs[1:]:
            a = a + r[0]
        pre = (jnp.dot(x, wa_ref[...], preferred_element_type=jnp.float32)
               + jnp.dot(a, wb_ref[...], preferred_element_type=jnp.float32)
               + b1_ref[...])
        nh = jnp.maximum(pre, 0.0)
        o_ref[...] = (
            jnp.dot(nh, w2_ref[...], preferred_element_type=jnp.float32)
            + b2_ref[...] + x)

    agg_specs = []
    for _ in aggs:
        agg_specs.append(pl.BlockSpec((1, _NB, H), lambda i: (0, i, 0)))
        agg_specs.append(pl.BlockSpec((1, _NB, H), lambda i: (1, i, 0)))
    agg_args = [a for a in aggs for _ in range(2)]
    return pl.pallas_call(
        body,
        grid=(n // _NB,),
        in_specs=(
            [pl.BlockSpec((_NB, H), lambda i: (i, 0))]
            + agg_specs
            + [pl.BlockSpec((H, H), lambda i: (0, 0))] * 3
            + [pl.BlockSpec((1, H), lambda i: (0, 0))] * 2
        ),
        out_specs=pl.BlockSpec((_NB, H), lambda i: (i, 0)),
        out_shape=jax.ShapeDtypeStruct((n, H), jnp.float32),
    )(x, *agg_args, wa, wb, w2, b1, b2)


def _sc_gather(u, v, src, dst):
    n_edges = src.shape[0]
    per_w = n_edges // _NW
    full = per_w // _CH
    rem = per_w - full * _CH
    groups = full // _RING
    assert groups * _RING == full
    mesh = plsc.VectorSubcoreMesh(core_axis_name="c", subcore_axis_name="s")

    @functools.partial(
        pl.kernel,
        mesh=mesh,
        out_type=jax.ShapeDtypeStruct((n_edges, H), jnp.float32),
        scratch_types=(
            [pltpu.VMEM((per_w,), jnp.int32)] * 2
            + [pltpu.VMEM((_CH, H), jnp.float32)] * (2 * _RING)
            + [pltpu.SemaphoreType.DMA] * (2 * _RING)
        ),
    )
    def k(u_hbm, v_hbm, src_hbm, dst_hbm, out_hbm, is_all, id_all, *rest):
        bufs_u = rest[0:_RING]
        bufs_v = rest[_RING:2 * _RING]
        sg = rest[2 * _RING:3 * _RING]
        sw = rest[3 * _RING:4 * _RING]
        wid = lax.axis_index("s") * 2 + lax.axis_index("c")
        w0 = wid * per_w

        pltpu.sync_copy(src_hbm.at[pl.ds(w0, per_w)], is_all)
        pltpu.sync_copy(dst_hbm.at[pl.ds(w0, per_w)], id_all)

        def issue_g(c, b):
            s = pl.ds(c * _CH, _CH)
            pltpu.async_copy(u_hbm.at[is_all.at[s]], bufs_u[b], sg[b])
            pltpu.async_copy(v_hbm.at[id_all.at[s]], bufs_v[b], sg[b])

        def wait_g(b):
            pltpu.make_async_copy(u_hbm.at[pl.ds(0, _CH)], bufs_u[b],
                                  sg[b]).wait()
            pltpu.make_async_copy(v_hbm.at[pl.ds(0, _CH)], bufs_v[b],
                                  sg[b]).wait()

        def add_uv(b, nrows):
            bu, bv = bufs_u[b], bufs_v[b]

            def row(r, carry):
                for cc in range(H // 16):
                    cs = pl.ds(cc * 16, 16)
                    bu[r, cs] = bu[r, cs] + bv[r, cs]
                return carry

            lax.fori_loop(0, nrows, row, 0)

        def issue_w(c, b):
            base = w0 + c * _CH
            pltpu.async_copy(bufs_u[b], out_hbm.at[pl.ds(base, _CH)], sw[b])

        def wait_w(b):
            pltpu.make_async_copy(bufs_u[b], out_hbm.at[pl.ds(0, _CH)],
                                  sw[b]).wait()

        def group(g, carry):
            for b in range(_RING):
                c = g * _RING + b
                pb = (b - 1) % _RING

                @pl.when(g > 0)
                def _():
                    wait_w(b)

                issue_g(c, b)
                if b == 0:
                    @pl.when(g > 0)
                    def _():
                        wait_g(pb)
                        add_uv(pb, _CH)
                        issue_w(c - 1, pb)
                else:
                    wait_g(pb)
                    add_uv(pb, _CH)
                    issue_w(c - 1, pb)
            return carry

        lax.fori_loop(0, groups, group, 0)
        lb = (full - 1) % _RING
        wait_g(lb)
        add_uv(lb, _CH)
        issue_w(full - 1, lb)
        for b in range(_RING):
            wait_w(b)
        if rem:
            base = full * _CH
            s = pl.ds(base, rem)
            d = pl.ds(0, rem)
            pltpu.async_copy(u_hbm.at[is_all.at[s]], bufs_u[0].at[d],
                             sg[0]).wait()
            pltpu.async_copy(v_hbm.at[id_all.at[s]], bufs_v[0].at[d],
                             sg[0]).wait()
            add_uv(0, rem)
            pltpu.sync_copy(bufs_u[0].at[d], out_hbm.at[pl.ds(w0 + base, rem)])

    return k(u, v, src, dst)


def _sc_scatter(parts, dst, zeros_nh, emit_ce=False):
    n = zeros_nh.shape[0]
    rows_per_tile = n // 16
    total = sum(arr.shape[0] for arr, _ in parts)
    mesh = plsc.VectorSubcoreMesh(core_axis_name="c", subcore_axis_name="s")

    meta = []
    for arr, off in parts:
        per_w = arr.shape[0] // _NW
        full = per_w // _CHS
        rem = per_w - full * _CHS
        assert full > 0 and full % _RING == 0
        meta.append((per_w, full, rem, off))
    rmax = max(max(m[2] for m in meta), 8)

    out_type = [jax.ShapeDtypeStruct((2, n, H), jnp.float32)]
    if emit_ce:
        out_type.append(jax.ShapeDtypeStruct((total, H), jnp.float32))

    @functools.partial(
        pl.kernel,
        mesh=mesh,
        out_type=tuple(out_type),
        scratch_types=(
            [pltpu.VMEM((_CHS,), jnp.int32)] * _RING
            + [pltpu.VMEM((_CHS, H), jnp.float32)] * _RING
            + [pltpu.VMEM((rmax,), jnp.int32),
               pltpu.VMEM((rmax, H), jnp.float32)]
            + [pltpu.VMEM_SHARED((n, H), jnp.float32)]
            + [pltpu.SemaphoreType.DMA] * (3 * _RING)
        ),
    )
    def k(*refs):
        np_ = len(parts)
        part_refs = refs[0:np_]
        dst_hbm = refs[np_]
        z_hbm = refs[np_ + 1]
        agg_hbm = refs[np_ + 2]
        pos = np_ + 3
        ce_hbm = refs[pos] if emit_ce else None
        pos += 1 if emit_ce else 0
        rest = refs[pos:]
        idx = rest[0:_RING]
        rows = rest[_RING:2 * _RING]
        idx_r = rest[2 * _RING]
        rows_r = rest[2 * _RING + 1]
        acc = rest[2 * _RING + 2]
        sl = rest[2 * _RING + 3:3 * _RING + 3]
        ss = rest[3 * _RING + 3:4 * _RING + 3]
        sc = rest[4 * _RING + 3:5 * _RING + 3]
        cid = lax.axis_index("c")
        sid = lax.axis_index("s")
        wid = sid * 2 + cid
        r0 = sid * rows_per_tile

        pltpu.sync_copy(z_hbm.at[pl.ds(r0, rows_per_tile)],
                        acc.at[pl.ds(r0, rows_per_tile)])
        plsc.subcore_barrier()

        for pi in range(np_):
            e_hbm = part_refs[pi]
            per_w, full, rem, off = meta[pi]
            w0l = wid * per_w
            groups = full // _RING

            def issue_l(c, b):
                bl = w0l + c * _CHS
                pltpu.async_copy(dst_hbm.at[pl.ds(off + bl, _CHS)], idx[b],
                                 sl[b])
                pltpu.async_copy(e_hbm.at[pl.ds(bl, _CHS)], rows[b], sl[b])

            def wait_l(b):
                pltpu.make_async_copy(dst_hbm.at[pl.ds(0, _CHS)], idx[b],
                                      sl[b]).wait()
                pltpu.make_async_copy(e_hbm.at[pl.ds(0, _CHS)], rows[b],
                                      sl[b]).wait()

            def issue_s(c, b):
                pltpu.async_copy(rows[b], acc.at[idx[b]], ss[b], add=True)
                if emit_ce:
                    base = off + w0l + c * _CHS
                    pltpu.async_copy(rows[b], ce_hbm.at[pl.ds(base, _CHS)],
                                     sc[b])

            def wait_s(b):
                pltpu.make_async_copy(rows[b], acc.at[pl.ds(0, _CHS)],
                                      ss[b]).wait()
                if emit_ce:
                    pltpu.make_async_copy(rows[b], ce_hbm.at[pl.ds(0, _CHS)],
                                          sc[b]).wait()

            def group(g, carry):
                for b in range(_RING):
                    c = g * _RING + b
                    pb = (b - 1) % _RING

                    @pl.when(g > 0)
                    def _():
                        wait_s(b)

                    issue_l(c, b)
                    if b == 0:
                        @pl.when(g > 0)
                        def _():
                            wait_l(pb)
                            issue_s(c - 1, pb)
                    else:
                        wait_l(pb)
                        issue_s(c - 1, pb)
                return carry

            lax.fori_loop(0, groups, group, 0)
            lb = (full - 1) % _RING
            wait_l(lb)
            issue_s(full - 1, lb)
            for b in range(_RING):
                wait_s(b)
            if rem:
                bl = w0l + full * _CHS
                rs = pl.ds(0, rem)
                pltpu.sync_copy(dst_hbm.at[pl.ds(off + bl, rem)],
                                idx_r.at[rs] if rem != rmax else idx_r)
                pltpu.sync_copy(e_hbm.at[pl.ds(bl, rem)],
                                rows_r.at[rs] if rem != rmax else rows_r)
                pltpu.sync_copy(rows_r.at[rs] if rem != rmax else rows_r,
                                acc.at[idx_r.at[rs] if rem != rmax else idx_r],
                                add=True)
                if emit_ce:
                    pltpu.sync_copy(
                        rows_r.at[rs] if rem != rmax else rows_r,
                        ce_hbm.at[pl.ds(off + bl, rem)])

        plsc.subcore_barrier()
        pltpu.sync_copy(acc.at[pl.ds(r0, rows_per_tile)],
                        agg_hbm.at[cid, pl.ds(r0, rows_per_tile)])

    out = k(*[arr for arr, _ in parts], dst, zeros_nh)
    return out if emit_ce else out[0]


def kernel(x, edge_attr, edge_index, params):
    src = edge_index[0]
    dst = edge_index[1]
    n = x.shape[0]
    ne = edge_attr.shape[0]
    el = ne // 2
    n_acc = ((n + 127) // 128) * 128
    zeros_nh = jnp.zeros((n_acc, H), dtype=jnp.float32)
    src_l, src_r = src[:el], src[el:]
    dst_l, dst_r = dst[:el], dst[el:]

    cx = x
    ce_l, ce_r = None, None
    ce_out = None
    for bi, p in enumerate(params):
        a = p['We1'][0:H]
        b = p['We1'][H:2 * H]
        c = p['We1'][2 * H:3 * H]
        w2 = p['We2']
        b1 = p['be1'].reshape(1, H)
        b2 = p['be2'].reshape(1, H)
        wa = p['Wn1'][0:H]
        wb = p['Wn1'][H:2 * H]
        n1 = p['bn1'].reshape(1, H)
        n2 = p['bn2'].reshape(1, H)
        last = bi == len(params) - 1

        u, v = _proj(cx, a, b)
        g_l = _sc_gather(u, v, src_l, dst_l)
        g_r = _sc_gather(u, v, src_r, dst_r)
        if bi == 0:
            e_l = _edge_mlp(g_l, edge_attr, 0, c, w2, b1, b2)
            e_r = _edge_mlp(g_r, edge_attr, el // _RB, c, w2, b1, b2)
        else:
            e_l = _edge_mlp(g_l, ce_l, 0, c, w2, b1, b2)
            e_r = _edge_mlp(g_r, ce_r, 0, c, w2, b1, b2)
        if last:
            agg, ce_out = _sc_scatter([(e_l, 0), (e_r, el)], dst, zeros_nh,
                                      emit_ce=True)
            aggs = [agg]
        else:
            agg_l = _sc_scatter([(e_l, 0)], dst, zeros_nh)
            agg_r = _sc_scatter([(e_r, el)], dst, zeros_nh)
            aggs = [agg_l, agg_r]
        cx = _node_mlp(cx, aggs, wa, wb, p['Wn2'], n1, n2)
        ce_l, ce_r = e_l, e_r

    return (cx, ce_out)
